# Initial kernel scaffold; baseline (speedup 1.0000x reference)
import functools

import jax
import jax.numpy as jnp
from jax import lax
from jax.experimental import pallas as pl
from jax.experimental.pallas import tpu as pltpu
from jax.experimental.pallas import tpu_sc as plsc

N = 10000
E = 160000
D = 256
D_OUT = 128
MLP_H = 256

NC = 2
NS = 16
L = 16

_MESH = plsc.VectorSubcoreMesh(
    core_axis_name="c", subcore_axis_name="s", num_cores=NC, num_subcores=NS
)


def _make_edge_pass(hc):
    e_per_tile = E // NS
    C = 80
    n_chunks = e_per_tile // C
    rows_per_tile = N // NS
    zrows = 125

    def body(table, scale, src, dst, out,
             src_v, dst_v, scale_v, src_c, dst_c, rows_v, zero_v, accum, sem):
        cid = lax.axis_index("c")
        sid = lax.axis_index("s")
        ebase = sid * e_per_tile

        pltpu.sync_copy(src.at[pl.ds(ebase, e_per_tile)], src_v)
        pltpu.sync_copy(dst.at[pl.ds(ebase, e_per_tile)], dst_v)
        pltpu.sync_copy(scale.at[pl.ds(ebase, e_per_tile)], scale_v)

        zero = jnp.zeros((L,), jnp.float32)

        def zbody(r, _):
            for j in range(hc // L):
                zero_v[r, pl.ds(j * L, L)] = zero
            return 0

        lax.fori_loop(0, zrows, zbody, 0)
        r0 = sid * rows_per_tile
        for t in range(rows_per_tile // zrows):
            pltpu.sync_copy(zero_v, accum.at[pl.ds(r0 + t * zrows, zrows)])
        plsc.subcore_barrier()

        off = cid * N

        def chunk(k, _):
            cb = k * C
            for j in range(C // L):
                sl = pl.ds(cb + j * L, L)
                src_c[pl.ds(j * L, L)] = src_v[sl] + off
                dst_c[pl.ds(j * L, L)] = dst_v[sl]
            pltpu.async_copy(table.at[src_c], rows_v, sem).wait()

            def ebody(e, _):
                sc = scale_v[cb + e]
                for j in range(hc // L):
                    sl = pl.ds(j * L, L)
                    rows_v[e, sl] = rows_v[e, sl] * sc
                return 0

            lax.fori_loop(0, C, ebody, 0)
            pltpu.sync_copy(rows_v, accum.at[dst_c], add=True)
            return 0

        lax.fori_loop(0, n_chunks, chunk, 0)
        plsc.subcore_barrier()
        pltpu.sync_copy(
            accum.at[pl.ds(r0, rows_per_tile)],
            out.at[pl.ds(cid * N + r0, rows_per_tile)],
        )

    return pl.kernel(
        body,
        out_type=jax.ShapeDtypeStruct((2 * N, hc), jnp.float32),
        mesh=_MESH,
        scratch_types=[
            pltpu.VMEM((e_per_tile,), jnp.int32),
            pltpu.VMEM((e_per_tile,), jnp.int32),
            pltpu.VMEM((e_per_tile,), jnp.float32),
            pltpu.VMEM((C,), jnp.int32),
            pltpu.VMEM((C,), jnp.int32),
            pltpu.VMEM((C, hc), jnp.float32),
            pltpu.VMEM((zrows, hc), jnp.float32),
            pltpu.VMEM_SHARED((N, hc), jnp.float32),
            pltpu.SemaphoreType.DMA,
        ],
        name=f"edge_pass_h{hc}",
    )


_edge_pass_16 = _make_edge_pass(16)
_edge_pass_128 = _make_edge_pass(128)
_edge_pass_64 = _make_edge_pass(64)


_E_PER_W = E // (NC * NS)


def _norm_body(dinv, w, src, dst, out, dinv_v, src_v, dst_v, w_v, norm_v):
    wid = lax.axis_index("s") * NC + lax.axis_index("c")
    base = wid * _E_PER_W
    pltpu.sync_copy(dinv, dinv_v)
    nv = _E_PER_W // L + 1
    zero_i = jnp.zeros((L,), jnp.int32)
    src_v[pl.ds(nv * L - L, L)] = zero_i
    dst_v[pl.ds(nv * L - L, L)] = zero_i
    pltpu.sync_copy(src.at[pl.ds(base, _E_PER_W)], src_v.at[pl.ds(0, _E_PER_W)])
    pltpu.sync_copy(dst.at[pl.ds(base, _E_PER_W)], dst_v.at[pl.ds(0, _E_PER_W)])
    pltpu.sync_copy(w.at[pl.ds(base, _E_PER_W)], w_v.at[pl.ds(0, _E_PER_W)])

    def vbody(i, _):
        sl = pl.ds(i * L, L)
        a = plsc.load_gather(dinv_v, [src_v[sl]])
        b = plsc.load_gather(dinv_v, [dst_v[sl]])
        norm_v[sl] = a * w_v[sl] * b
        return 0

    lax.fori_loop(0, nv, vbody, 0)
    pltpu.sync_copy(norm_v.at[pl.ds(0, _E_PER_W)], out.at[pl.ds(base, _E_PER_W)])


_norm_kernel = pl.kernel(
    _norm_body,
    out_type=jax.ShapeDtypeStruct((E,), jnp.float32),
    mesh=_MESH,
    scratch_types=[
        pltpu.VMEM((N,), jnp.float32),
        pltpu.VMEM((_E_PER_W + L,), jnp.int32),
        pltpu.VMEM((_E_PER_W + L,), jnp.int32),
        pltpu.VMEM((_E_PER_W + L,), jnp.float32),
        pltpu.VMEM((_E_PER_W + L,), jnp.float32),
    ],
    name="edge_norm",
)


def _ef_body(emb, src, dst, out_s, out_d, src_v, dst_v, rows_s, rows_d,
             sem_s, sem_d):
    C = 80
    wid = lax.axis_index("s") * NC + lax.axis_index("c")
    base = wid * _E_PER_W
    pltpu.sync_copy(src.at[pl.ds(base, _E_PER_W)], src_v)
    pltpu.sync_copy(dst.at[pl.ds(base, _E_PER_W)], dst_v)
    n_chunks = (_E_PER_W + C - 1) // C

    def chunk(k, _):
        cb = jnp.minimum(k * C, _E_PER_W - C)
        cs = pltpu.async_copy(emb.at[src_v.at[pl.ds(cb, C)]], rows_s, sem_s)
        cd = pltpu.async_copy(emb.at[dst_v.at[pl.ds(cb, C)]], rows_d, sem_d)
        cs.wait()
        cd.wait()
        pltpu.sync_copy(rows_s, out_s.at[pl.ds(base + cb, C)])
        pltpu.sync_copy(rows_d, out_d.at[pl.ds(base + cb, C)])
        return 0

    lax.fori_loop(0, n_chunks, chunk, 0)


_ef_kernel = pl.kernel(
    _ef_body,
    out_type=(
        jax.ShapeDtypeStruct((E, D_OUT), jnp.float32),
        jax.ShapeDtypeStruct((E, D_OUT), jnp.float32),
    ),
    mesh=_MESH,
    scratch_types=[
        pltpu.VMEM((_E_PER_W,), jnp.int32),
        pltpu.VMEM((_E_PER_W,), jnp.int32),
        pltpu.VMEM((80, D_OUT), jnp.float32),
        pltpu.VMEM((80, D_OUT), jnp.float32),
        pltpu.SemaphoreType.DMA,
        pltpu.SemaphoreType.DMA,
    ],
    name="ef_gather",
)


_BN = 400
_NB = N // _BN


def _dinv_body(deg_ref, dinv_ref, dinv2_ref):
    deg = deg_ref[:, 0:1] + 1.0
    dinv = jnp.where(deg > 0, lax.rsqrt(deg), 0.0)
    dinv_ref[...] = dinv
    dinv2_ref[...] = dinv * dinv


def _dinv_kernel(deg16):
    return pl.pallas_call(
        _dinv_body,
        grid=(_NB,),
        in_specs=[pl.BlockSpec((_BN, 16), lambda i: (i, 0))],
        out_specs=(
            pl.BlockSpec((_BN, 1), lambda i: (i, 0)),
            pl.BlockSpec((_BN, 1), lambda i: (i, 0)),
        ),
        out_shape=(
            jax.ShapeDtypeStruct((N, 1), jnp.float32),
            jax.ShapeDtypeStruct((N, 1), jnp.float32),
        ),
        name="dinv",
    )(deg16)


def _mm1_body(x_ref, w_ref, out_ref):
    out_ref[...] = jnp.dot(x_ref[...], w_ref[...],
                           preferred_element_type=jnp.float32)


def _mm1(x, W1):
    return pl.pallas_call(
        _mm1_body,
        grid=(_NB, NC),
        in_specs=[
            pl.BlockSpec((_BN, D), lambda i, c: (i, 0)),
            pl.BlockSpec((D, D // 2), lambda i, c: (0, c)),
        ],
        out_specs=pl.BlockSpec((_BN, D // 2), lambda i, c: (c * _NB + i, 0)),
        out_shape=jax.ShapeDtypeStruct((2 * N, D // 2), jnp.float32),
        name="mm1",
    )(x, W1)


def _mm_next_body(hw, sA_ref, sB_ref, gA_ref, gB_ref, d2_ref, b_ref, w_ref,
                  out_ref):
    d2 = d2_ref[...]
    b = b_ref[...]
    hA = jnp.maximum(sA_ref[...] + d2 * gA_ref[...] + b[:, :hw], 0.0)
    hB = jnp.maximum(sB_ref[...] + d2 * gB_ref[...] + b[:, hw:], 0.0)
    out_ref[...] = (
        jnp.dot(hA, w_ref[:hw, :], preferred_element_type=jnp.float32)
        + jnp.dot(hB, w_ref[hw:, :], preferred_element_type=jnp.float32)
    )


def _mm_next(s, g, dinv2, b, W, h_in, h_out):
    hw = h_in // 2
    ow = h_out // 2
    return pl.pallas_call(
        functools.partial(_mm_next_body, hw),
        grid=(_NB, NC),
        in_specs=[
            pl.BlockSpec((_BN, hw), lambda i, c: (i, 0)),
            pl.BlockSpec((_BN, hw), lambda i, c: (_NB + i, 0)),
            pl.BlockSpec((_BN, hw), lambda i, c: (i, 0)),
            pl.BlockSpec((_BN, hw), lambda i, c: (_NB + i, 0)),
            pl.BlockSpec((_BN, 1), lambda i, c: (i, 0)),
            pl.BlockSpec((1, h_in), lambda i, c: (0, 0)),
            pl.BlockSpec((h_in, ow), lambda i, c: (0, c)),
        ],
        out_specs=pl.BlockSpec((_BN, ow), lambda i, c: (c * _NB + i, 0)),
        out_shape=jax.ShapeDtypeStruct((2 * N, ow), jnp.float32),
        name="mm_next",
    )(s, s, g, g, dinv2, b, W)


def _emb_body(sA_ref, sB_ref, gA_ref, gB_ref, d2_ref, b_ref, xout_ref, emb_ref):
    d2 = d2_ref[...]
    b = b_ref[...]
    hw = D_OUT // 2
    xA = sA_ref[...] + d2 * gA_ref[...] + b[:, :hw]
    xB = sB_ref[...] + d2 * gB_ref[...] + b[:, hw:]
    xo = jnp.concatenate([xA, xB], axis=1)
    xout_ref[...] = xo
    emb_ref[...] = jnp.maximum(xo, 0.0)


def _emb_kernel(s3, g3, dinv2, b3):
    hw = D_OUT // 2
    return pl.pallas_call(
        _emb_body,
        grid=(_NB,),
        in_specs=[
            pl.BlockSpec((_BN, hw), lambda i: (i, 0)),
            pl.BlockSpec((_BN, hw), lambda i: (_NB + i, 0)),
            pl.BlockSpec((_BN, hw), lambda i: (i, 0)),
            pl.BlockSpec((_BN, hw), lambda i: (_NB + i, 0)),
            pl.BlockSpec((_BN, 1), lambda i: (i, 0)),
            pl.BlockSpec((1, D_OUT), lambda i: (0, 0)),
        ],
        out_specs=(
            pl.BlockSpec((_BN, D_OUT), lambda i: (i, 0)),
            pl.BlockSpec((_BN, D_OUT), lambda i: (i, 0)),
        ),
        out_shape=(
            jax.ShapeDtypeStruct((N, D_OUT), jnp.float32),
            jax.ShapeDtypeStruct((N, D_OUT), jnp.float32),
        ),
        name="emb",
    )(s3, s3, g3, g3, dinv2, b3)


_BE = 640
_EB = E // _BE


def _mlp_body(efs_ref, efd_ref, w_ref, m1a_ref, m1r_ref, m1b_ref, mb1_ref,
              m2_ref, mb2_ref, m3_ref, mb3_ref, out_ref):
    m = (
        jnp.dot(efs_ref[...], m1a_ref[...], preferred_element_type=jnp.float32)
        + jnp.dot(efd_ref[...], m1b_ref[...], preferred_element_type=jnp.float32)
        + w_ref[...] * m1r_ref[...]
        + mb1_ref[...]
    )
    m = jnp.maximum(m, 0.0)
    m = jnp.dot(m, m2_ref[...], preferred_element_type=jnp.float32) + mb2_ref[...]
    m = jnp.maximum(m, 0.0)
    out_ref[...] = (
        jnp.dot(m, m3_ref[...], preferred_element_type=jnp.float32) + mb3_ref[...]
    )


def _mlp_kernel(efs, efd, ew, M1a, M1r, M1b, mb1, M2, mb2, M3, mb3):
    def full(shape):
        return pl.BlockSpec(shape, lambda i: tuple(0 for _ in shape))

    return pl.pallas_call(
        _mlp_body,
        grid=(_EB,),
        in_specs=[
            pl.BlockSpec((_BE, D_OUT), lambda i: (i, 0)),
            pl.BlockSpec((_BE, D_OUT), lambda i: (i, 0)),
            pl.BlockSpec((_BE, 1), lambda i: (i, 0)),
            full((D_OUT, MLP_H)),
            full((1, MLP_H)),
            full((D_OUT, MLP_H)),
            full((1, MLP_H)),
            full((MLP_H, MLP_H)),
            full((1, MLP_H)),
            full((MLP_H, 2)),
            full((1, 2)),
        ],
        out_specs=pl.BlockSpec((_BE, 2), lambda i: (i, 0)),
        out_shape=jax.ShapeDtypeStruct((E, 2), jnp.float32),
        name="edge_mlp",
    )(efs, efd, ew, M1a, M1r, M1b, mb1, M2, mb2, M3, mb3)


def kernel(x, edge_index, edge_weight, W1, b1, W2, b2, W3, b3,
           M1, mb1, M2, mb2, M3, mb3):
    src = edge_index[0]
    dst = edge_index[1]
    w = edge_weight[:, 0]

    ones_tab = jnp.ones((2 * N, 16), jnp.float32)
    deg16 = _edge_pass_16(ones_tab, w, src, dst)
    dinv, dinv2 = _dinv_kernel(deg16)
    norm = _norm_kernel(dinv[:, 0], w, src, dst)

    g1 = _mm1(x, W1)
    s1 = _edge_pass_128(g1, norm, src, dst)
    g2 = _mm_next(s1, g1, dinv2, b1.reshape(1, -1), W2, D, D)
    s2 = _edge_pass_128(g2, norm, src, dst)
    g3 = _mm_next(s2, g2, dinv2, b2.reshape(1, -1), W3, D, D_OUT)
    s3 = _edge_pass_64(g3, norm, src, dst)
    x_out, emb = _emb_kernel(s3, g3, dinv2, b3.reshape(1, -1))

    efs, efd = _ef_kernel(emb, src, dst)
    edge_out = _mlp_kernel(
        efs, efd, edge_weight,
        M1[:D_OUT], M1[D_OUT:D_OUT + 1], M1[D_OUT + 1:],
        mb1.reshape(1, -1), M2, mb2.reshape(1, -1), M3, mb3.reshape(1, -1),
    )
    return (x_out, edge_out)


# speedup vs baseline: 5.6060x; 5.6060x over previous
import functools

import jax
import jax.numpy as jnp
from jax import lax
from jax.experimental import pallas as pl
from jax.experimental.pallas import tpu as pltpu
from jax.experimental.pallas import tpu_sc as plsc

N = 10000
E = 160000
EP = 163840
D = 256
D_OUT = 128
MLP_H = 256

NC = 2
NS = 16
L = 16
NP = 10240

_MESH = plsc.VectorSubcoreMesh(
    core_axis_name="c", subcore_axis_name="s", num_cores=NC, num_subcores=NS
)


HC = 128


def _make_edge_pass(mode):
    per_core = (mode == "feat")
    e_per_tile = EP // NS if per_core else EP // (NC * NS)
    C = 80
    n_chunks = e_per_tile // C
    rows_per_tile = NP // NS
    zrows = 32
    gather = (mode != "deg")

    def body(*refs):
        if gather:
            table, scale, src, dst, out = refs[:5]
            scr = refs[5:]
        else:
            scale, src, dst, out = refs[:4]
            scr = refs[4:]
        src_v, dst_v, scale_v, src_c, dst_c, rows_v, zero_v, accum, sem = scr
        cid = lax.axis_index("c")
        sid = lax.axis_index("s")
        if per_core:
            ebase = sid * e_per_tile
        else:
            ebase = (cid * NS + sid) * e_per_tile

        pltpu.sync_copy(src.at[pl.ds(ebase, e_per_tile)], src_v)
        pltpu.sync_copy(dst.at[pl.ds(ebase, e_per_tile)], dst_v)
        pltpu.sync_copy(scale.at[pl.ds(ebase, e_per_tile)], scale_v)

        zero = jnp.zeros((L,), jnp.float32)

        def zbody(r, _):
            for j in range(HC // L):
                zero_v[r, pl.ds(j * L, L)] = zero
            return 0

        lax.fori_loop(0, zrows, zbody, 0)
        r0 = sid * rows_per_tile
        for t in range(rows_per_tile // zrows):
            pltpu.sync_copy(zero_v, accum.at[pl.ds(r0 + t * zrows, zrows)])
        plsc.subcore_barrier()

        off = cid * N

        def chunk(k, _):
            cb = k * C
            for j in range(C // L):
                sl = pl.ds(cb + j * L, L)
                if gather:
                    idx = src_v[sl]
                    src_c[pl.ds(j * L, L)] = idx + off if per_core else idx
                dst_c[pl.ds(j * L, L)] = dst_v[sl]
            if gather:
                pltpu.async_copy(table.at[src_c], rows_v, sem).wait()

            def ebody(m, _):
                sv = scale_v[pl.ds(cb + m * L, L)]
                for e in range(L):
                    sc = sv[e]
                    for j in range(HC // L):
                        sl = pl.ds(j * L, L)
                        if gather:
                            rows_v[m * L + e, sl] = rows_v[m * L + e, sl] * sc
                        else:
                            rows_v[m * L + e, sl] = jnp.full((L,), sc)
                return 0

            lax.fori_loop(0, C // L, ebody, 0)
            pltpu.sync_copy(rows_v, accum.at[dst_c], add=True)
            return 0

        lax.fori_loop(0, n_chunks, chunk, 0)
        plsc.subcore_barrier()
        pltpu.sync_copy(
            accum.at[pl.ds(r0, rows_per_tile)],
            out.at[cid, pl.ds(r0, rows_per_tile)],
        )

    return pl.kernel(
        body,
        out_type=jax.ShapeDtypeStruct((NC, NP, HC), jnp.float32),
        mesh=_MESH,
        scratch_types=[
            pltpu.VMEM((e_per_tile,), jnp.int32),
            pltpu.VMEM((e_per_tile,), jnp.int32),
            pltpu.VMEM((e_per_tile,), jnp.float32),
            pltpu.VMEM((C,), jnp.int32),
            pltpu.VMEM((C,), jnp.int32),
            pltpu.VMEM((C, HC), jnp.float32),
            pltpu.VMEM((zrows, HC), jnp.float32),
            pltpu.VMEM_SHARED((NP, HC), jnp.float32),
            pltpu.SemaphoreType.DMA,
        ],
        name=f"edge_pass_{mode}",
    )


_edge_pass_feat = _make_edge_pass("feat")
_edge_pass_edge = _make_edge_pass("edge")
_edge_pass_deg = _make_edge_pass("deg")


_E_PER_W = EP // (NC * NS)


def _ef_body(emb, src, dst, out_s, out_d, src_v, dst_v, rows_s, rows_d,
             sem_s, sem_d):
    C = 80
    wid = lax.axis_index("c") * NS + lax.axis_index("s")
    base = wid * _E_PER_W
    pltpu.sync_copy(src.at[pl.ds(base, _E_PER_W)], src_v)
    pltpu.sync_copy(dst.at[pl.ds(base, _E_PER_W)], dst_v)

    def chunk(k, _):
        cb = k * C
        cs = pltpu.async_copy(emb.at[src_v.at[pl.ds(cb, C)]], rows_s, sem_s)
        cd = pltpu.async_copy(emb.at[dst_v.at[pl.ds(cb, C)]], rows_d, sem_d)
        cs.wait()
        cd.wait()
        pltpu.sync_copy(rows_s, out_s.at[pl.ds(base + cb, C)])
        pltpu.sync_copy(rows_d, out_d.at[pl.ds(base + cb, C)])
        return 0

    lax.fori_loop(0, _E_PER_W // C, chunk, 0)


_ef_kernel = pl.kernel(
    _ef_body,
    out_type=(
        jax.ShapeDtypeStruct((EP, D_OUT), jnp.float32),
        jax.ShapeDtypeStruct((EP, D_OUT), jnp.float32),
    ),
    mesh=_MESH,
    scratch_types=[
        pltpu.VMEM((_E_PER_W,), jnp.int32),
        pltpu.VMEM((_E_PER_W,), jnp.int32),
        pltpu.VMEM((80, D_OUT), jnp.float32),
        pltpu.VMEM((80, D_OUT), jnp.float32),
        pltpu.SemaphoreType.DMA,
        pltpu.SemaphoreType.DMA,
    ],
    name="ef_gather",
)


_BN = 400
_NB = N // _BN


def _dinv_body(deg_ref, dinv_ref):
    deg = deg_ref[0, :, 0:1] + deg_ref[1, :, 0:1] + 1.0
    dinv_ref[...] = jnp.where(deg > 0, lax.rsqrt(deg), 0.0)


def _dinv_kernel(deg):
    return pl.pallas_call(
        _dinv_body,
        grid=(_NB,),
        in_specs=[pl.BlockSpec((2, _BN, HC), lambda i: (0, i, 0))],
        out_specs=pl.BlockSpec((_BN, 1), lambda i: (i, 0)),
        out_shape=jax.ShapeDtypeStruct((N, 1), jnp.float32),
        name="dinv",
    )(deg)


def _mm1_body(x_ref, w_ref, d_ref, out_ref):
    out_ref[...] = d_ref[...] * jnp.dot(x_ref[...], w_ref[...],
                                        preferred_element_type=jnp.float32)


def _mm1(x, W1, dinv):
    return pl.pallas_call(
        _mm1_body,
        grid=(_NB, NC),
        in_specs=[
            pl.BlockSpec((_BN, D), lambda i, c: (i, 0)),
            pl.BlockSpec((D, D // 2), lambda i, c: (0, c)),
            pl.BlockSpec((_BN, 1), lambda i, c: (i, 0)),
        ],
        out_specs=pl.BlockSpec((_BN, D // 2), lambda i, c: (c * _NB + i, 0)),
        out_shape=jax.ShapeDtypeStruct((2 * N, D // 2), jnp.float32),
        name="mm1",
    )(x, W1, dinv)


def _mm_next_body(hw, sA_ref, sB_ref, gA_ref, gB_ref, d_ref, b_ref, w_ref,
                  out_ref):
    d = d_ref[...]
    b = b_ref[...]
    hA = jnp.maximum(d * (sA_ref[0] + gA_ref[...]) + b[:, :hw], 0.0)
    hB = jnp.maximum(d * (sB_ref[0] + gB_ref[...]) + b[:, hw:], 0.0)
    out_ref[...] = d * (
        jnp.dot(hA, w_ref[:hw, :], preferred_element_type=jnp.float32)
        + jnp.dot(hB, w_ref[hw:, :], preferred_element_type=jnp.float32)
    )


def _mm_next(s, g, dinv, b, W, h_in, h_out, split_out=True):
    hw = h_in // 2
    if split_out:
        ow = h_out // 2
        return pl.pallas_call(
            functools.partial(_mm_next_body, hw),
            grid=(_NB, NC),
            in_specs=[
                pl.BlockSpec((1, _BN, hw), lambda i, c: (0, i, 0)),
                pl.BlockSpec((1, _BN, hw), lambda i, c: (1, i, 0)),
                pl.BlockSpec((_BN, hw), lambda i, c: (i, 0)),
                pl.BlockSpec((_BN, hw), lambda i, c: (_NB + i, 0)),
                pl.BlockSpec((_BN, 1), lambda i, c: (i, 0)),
                pl.BlockSpec((1, h_in), lambda i, c: (0, 0)),
                pl.BlockSpec((h_in, ow), lambda i, c: (0, c)),
            ],
            out_specs=pl.BlockSpec((_BN, ow), lambda i, c: (c * _NB + i, 0)),
            out_shape=jax.ShapeDtypeStruct((2 * N, ow), jnp.float32),
            name="mm_next",
        )(s, s, g, g, dinv, b, W)
    return pl.pallas_call(
        functools.partial(_mm_next_body, hw),
        grid=(_NB,),
        in_specs=[
            pl.BlockSpec((1, _BN, hw), lambda i: (0, i, 0)),
            pl.BlockSpec((1, _BN, hw), lambda i: (1, i, 0)),
            pl.BlockSpec((_BN, hw), lambda i: (i, 0)),
            pl.BlockSpec((_BN, hw), lambda i: (_NB + i, 0)),
            pl.BlockSpec((_BN, 1), lambda i: (i, 0)),
            pl.BlockSpec((1, h_in), lambda i: (0, 0)),
            pl.BlockSpec((h_in, h_out), lambda i: (0, 0)),
        ],
        out_specs=pl.BlockSpec((_BN, h_out), lambda i: (i, 0)),
        out_shape=jax.ShapeDtypeStruct((N, h_out), jnp.float32),
        name="mm_next_full",
    )(s, s, g, g, dinv, b, W)


def _emb_body(s0_ref, s1_ref, g_ref, d_ref, b_ref, xout_ref, emb_ref):
    xo = d_ref[...] * (s0_ref[0] + s1_ref[0] + g_ref[...]) + b_ref[...]
    xout_ref[...] = xo
    emb_ref[...] = jnp.maximum(xo, 0.0)


def _emb_kernel(s3, g3, dinv, b3):
    return pl.pallas_call(
        _emb_body,
        grid=(_NB,),
        in_specs=[
            pl.BlockSpec((1, _BN, D_OUT), lambda i: (0, i, 0)),
            pl.BlockSpec((1, _BN, D_OUT), lambda i: (1, i, 0)),
            pl.BlockSpec((_BN, D_OUT), lambda i: (i, 0)),
            pl.BlockSpec((_BN, 1), lambda i: (i, 0)),
            pl.BlockSpec((1, D_OUT), lambda i: (0, 0)),
        ],
        out_specs=(
            pl.BlockSpec((_BN, D_OUT), lambda i: (i, 0)),
            pl.BlockSpec((_BN, D_OUT), lambda i: (i, 0)),
        ),
        out_shape=(
            jax.ShapeDtypeStruct((N, D_OUT), jnp.float32),
            jax.ShapeDtypeStruct((N, D_OUT), jnp.float32),
        ),
        name="emb",
    )(s3, s3, g3, dinv, b3)


_BE = 640
_EB = E // _BE


def _mlp_body(efs_ref, efd_ref, w_ref, m1a_ref, m1r_ref, m1b_ref, mb1_ref,
              m2_ref, mb2_ref, m3_ref, mb3_ref, out_ref):
    m = (
        jnp.dot(efs_ref[...], m1a_ref[...], preferred_element_type=jnp.float32)
        + jnp.dot(efd_ref[...], m1b_ref[...], preferred_element_type=jnp.float32)
        + w_ref[...] * m1r_ref[...]
        + mb1_ref[...]
    )
    m = jnp.maximum(m, 0.0)
    m = jnp.dot(m, m2_ref[...], preferred_element_type=jnp.float32) + mb2_ref[...]
    m = jnp.maximum(m, 0.0)
    out_ref[...] = (
        jnp.dot(m, m3_ref[...], preferred_element_type=jnp.float32) + mb3_ref[...]
    )


def _mlp_kernel(efs, efd, ew, M1a, M1r, M1b, mb1, M2, mb2, M3, mb3):
    def full(shape):
        return pl.BlockSpec(shape, lambda i: tuple(0 for _ in shape))

    return pl.pallas_call(
        _mlp_body,
        grid=(_EB,),
        in_specs=[
            pl.BlockSpec((_BE, D_OUT), lambda i: (i, 0)),
            pl.BlockSpec((_BE, D_OUT), lambda i: (i, 0)),
            pl.BlockSpec((_BE, 1), lambda i: (i, 0)),
            full((D_OUT, MLP_H)),
            full((1, MLP_H)),
            full((D_OUT, MLP_H)),
            full((1, MLP_H)),
            full((MLP_H, MLP_H)),
            full((1, MLP_H)),
            full((MLP_H, 2)),
            full((1, 2)),
        ],
        out_specs=pl.BlockSpec((_BE, 2), lambda i: (i, 0)),
        out_shape=jax.ShapeDtypeStruct((E, 2), jnp.float32),
        name="edge_mlp",
    )(efs, efd, ew, M1a, M1r, M1b, mb1, M2, mb2, M3, mb3)


def kernel(x, edge_index, edge_weight, W1, b1, W2, b2, W3, b3,
           M1, mb1, M2, mb2, M3, mb3):
    npad = EP - E
    pad_idx = (jnp.arange(npad, dtype=jnp.int32) * 97) % N
    src = jnp.concatenate([edge_index[0], pad_idx])
    dst = jnp.concatenate([edge_index[1], pad_idx])
    w = jnp.concatenate([edge_weight[:, 0], jnp.zeros((npad,), jnp.float32)])

    deg = _edge_pass_deg(w, src, dst)
    dinv = _dinv_kernel(deg)

    g1 = _mm1(x, W1, dinv)
    s1 = _edge_pass_feat(g1, w, src, dst)
    g2 = _mm_next(s1, g1, dinv, b1.reshape(1, -1), W2, D, D)
    s2 = _edge_pass_feat(g2, w, src, dst)
    g3 = _mm_next(s2, g2, dinv, b2.reshape(1, -1), W3, D, D_OUT,
                  split_out=False)
    s3 = _edge_pass_edge(g3, w, src, dst)
    x_out, emb = _emb_kernel(s3, g3, dinv, b3.reshape(1, -1))

    efs, efd = _ef_kernel(emb, src, dst)
    edge_out = _mlp_kernel(
        efs, efd, edge_weight,
        M1[:D_OUT], M1[D_OUT:D_OUT + 1], M1[D_OUT + 1:],
        mb1.reshape(1, -1), M2, mb2.reshape(1, -1), M3, mb3.reshape(1, -1),
    )
    return (x_out, edge_out)


# speedup vs baseline: 7.2921x; 1.3008x over previous
import functools

import jax
import jax.numpy as jnp
from jax import lax
from jax.experimental import pallas as pl
from jax.experimental.pallas import tpu as pltpu
from jax.experimental.pallas import tpu_sc as plsc

N = 10000
E = 160000
EP = 163840
D = 256
D_OUT = 128
MLP_H = 256

NC = 2
NS = 16
L = 16
NP = 10240

_MESH = plsc.VectorSubcoreMesh(
    core_axis_name="c", subcore_axis_name="s", num_cores=NC, num_subcores=NS
)


HC = 128


def _make_edge_pass(mode):
    per_core = (mode == "feat")
    e_per_tile = EP // NS if per_core else EP // (NC * NS)
    C = 64
    n_chunks = e_per_tile // C
    rows_per_tile = NP // NS
    zrows = 64
    gather = (mode != "deg")

    def body(*refs):
        if gather:
            table, scale, src, dst, out = refs[:5]
            scr = refs[5:]
        else:
            scale, src, dst, out = refs[:4]
            scr = refs[4:]
        (src_v, dst_v, scale_v, src_c0, src_c1, dst_c0, dst_c1, rows0, rows1,
         accum, gsem0, gsem1, ssem0, ssem1, zsem) = scr
        src_c = (src_c0, src_c1)
        dst_c = (dst_c0, dst_c1)
        rows = (rows0, rows1)
        gsem = (gsem0, gsem1)
        ssem = (ssem0, ssem1)
        cid = lax.axis_index("c")
        sid = lax.axis_index("s")
        if per_core:
            ebase = sid * e_per_tile
        else:
            ebase = (cid * NS + sid) * e_per_tile

        zero = jnp.zeros((L,), jnp.float32)

        def zbody(r, _):
            for j in range(HC // L):
                rows0[r, pl.ds(j * L, L)] = zero
            return 0

        lax.fori_loop(0, zrows, zbody, 0)
        r0 = sid * rows_per_tile
        for t in range(rows_per_tile // zrows):
            pltpu.make_async_copy(
                rows0, accum.at[pl.ds(r0 + t * zrows, zrows)], zsem).start()

        if gather:
            pltpu.sync_copy(src.at[pl.ds(ebase, e_per_tile)], src_v)
        pltpu.sync_copy(dst.at[pl.ds(ebase, e_per_tile)], dst_v)
        pltpu.sync_copy(scale.at[pl.ds(ebase, e_per_tile)], scale_v)

        for t in range(rows_per_tile // zrows):
            pltpu.make_async_copy(
                rows0, accum.at[pl.ds(r0 + t * zrows, zrows)], zsem).wait()
        plsc.subcore_barrier()

        off = cid * N

        def prep(k, b):
            cb = k * C
            for j in range(C // L):
                sl = pl.ds(cb + j * L, L)
                if gather:
                    idx = src_v[sl]
                    src_c[b][pl.ds(j * L, L)] = idx + off if per_core else idx
                dst_c[b][pl.ds(j * L, L)] = dst_v[sl]

        def gissue(b):
            pltpu.make_async_copy(table.at[src_c[b]], rows[b], gsem[b]).start()

        def gwait(b):
            pltpu.make_async_copy(table.at[src_c[b]], rows[b], gsem[b]).wait()

        def sissue(b):
            pltpu.async_copy(rows[b], accum.at[dst_c[b]], ssem[b], add=True)

        def swait(b):
            pltpu.make_async_copy(rows[b], accum.at[dst_c[b]], ssem[b]).wait()

        def scale_chunk(k, b):
            cb = k * C

            def ebody(m, _):
                sv = scale_v[pl.ds(cb + m * L, L)]
                for e in range(L):
                    sc = sv[e]
                    for j in range(HC // L):
                        sl = pl.ds(j * L, L)
                        if gather:
                            rows[b][m * L + e, sl] = rows[b][m * L + e, sl] * sc
                        else:
                            rows[b][m * L + e, sl] = jnp.full((L,), sc)
                return 0

            lax.fori_loop(0, C // L, ebody, 0)

        if gather:
            prep(0, 0)
            gissue(0)
            prep(1, 1)
            gissue(1)
            gwait(0)
            scale_chunk(0, 0)
            sissue(0)

            def pair(p, _):
                k1 = 2 * p - 1
                swait(0)
                prep(k1 + 1, 0)
                gissue(0)
                gwait(1)
                scale_chunk(k1, 1)
                sissue(1)
                k2 = 2 * p
                swait(1)
                prep(k2 + 1, 1)
                gissue(1)
                gwait(0)
                scale_chunk(k2, 0)
                sissue(0)
                return 0

            lax.fori_loop(1, n_chunks // 2, pair, 0)
            gwait(1)
            scale_chunk(n_chunks - 1, 1)
            sissue(1)
            swait(0)
            swait(1)
        else:
            prep(0, 0)
            scale_chunk(0, 0)
            sissue(0)
            prep(1, 1)
            scale_chunk(1, 1)
            sissue(1)

            def pair(p, _):
                k1 = 2 * p
                swait(0)
                prep(k1, 0)
                scale_chunk(k1, 0)
                sissue(0)
                k2 = 2 * p + 1
                swait(1)
                prep(k2, 1)
                scale_chunk(k2, 1)
                sissue(1)
                return 0

            lax.fori_loop(1, n_chunks // 2, pair, 0)
            swait(0)
            swait(1)

        plsc.subcore_barrier()
        pltpu.sync_copy(
            accum.at[pl.ds(r0, rows_per_tile)],
            out.at[cid, pl.ds(r0, rows_per_tile)],
        )

    return pl.kernel(
        body,
        out_type=jax.ShapeDtypeStruct((NC, NP, HC), jnp.float32),
        mesh=_MESH,
        scratch_types=[
            pltpu.VMEM((e_per_tile,), jnp.int32),
            pltpu.VMEM((e_per_tile,), jnp.int32),
            pltpu.VMEM((e_per_tile,), jnp.float32),
            pltpu.VMEM((C,), jnp.int32),
            pltpu.VMEM((C,), jnp.int32),
            pltpu.VMEM((C,), jnp.int32),
            pltpu.VMEM((C,), jnp.int32),
            pltpu.VMEM((C, HC), jnp.float32),
            pltpu.VMEM((C, HC), jnp.float32),
            pltpu.VMEM_SHARED((NP, HC), jnp.float32),
            pltpu.SemaphoreType.DMA,
            pltpu.SemaphoreType.DMA,
            pltpu.SemaphoreType.DMA,
            pltpu.SemaphoreType.DMA,
            pltpu.SemaphoreType.DMA,
        ],
        name=f"edge_pass_{mode}",
    )


_edge_pass_feat = _make_edge_pass("feat")
_edge_pass_edge = _make_edge_pass("edge")
_edge_pass_deg = _make_edge_pass("deg")


_E_PER_W = EP // (NC * NS)


def _ef_body(emb, src, dst, out_s, out_d, src_v, dst_v,
             rs0, rs1, rd0, rd1, gsem0, gsem1, wsem0, wsem1):
    C = 128
    n_chunks = _E_PER_W // C
    wid = lax.axis_index("c") * NS + lax.axis_index("s")
    base = wid * _E_PER_W
    pltpu.sync_copy(src.at[pl.ds(base, _E_PER_W)], src_v)
    pltpu.sync_copy(dst.at[pl.ds(base, _E_PER_W)], dst_v)
    rs = (rs0, rs1)
    rd = (rd0, rd1)
    gsem = (gsem0, gsem1)
    wsem = (wsem0, wsem1)

    def gissue(k, b):
        cb = k * C
        pltpu.make_async_copy(
            emb.at[src_v.at[pl.ds(cb, C)]], rs[b], gsem[b]).start()
        pltpu.make_async_copy(
            emb.at[dst_v.at[pl.ds(cb, C)]], rd[b], gsem[b]).start()

    def gwait(k, b):
        cb = k * C
        pltpu.make_async_copy(
            emb.at[src_v.at[pl.ds(cb, C)]], rs[b], gsem[b]).wait()
        pltpu.make_async_copy(
            emb.at[dst_v.at[pl.ds(cb, C)]], rd[b], gsem[b]).wait()

    def wissue(k, b):
        cb = k * C
        pltpu.make_async_copy(rs[b], out_s.at[pl.ds(base + cb, C)],
                              wsem[b]).start()
        pltpu.make_async_copy(rd[b], out_d.at[pl.ds(base + cb, C)],
                              wsem[b]).start()

    def wwait(k, b):
        cb = k * C
        pltpu.make_async_copy(rs[b], out_s.at[pl.ds(base + cb, C)],
                              wsem[b]).wait()
        pltpu.make_async_copy(rd[b], out_d.at[pl.ds(base + cb, C)],
                              wsem[b]).wait()

    gissue(0, 0)
    gissue(1, 1)
    gwait(0, 0)
    wissue(0, 0)

    def pair(p, _):
        k1 = 2 * p - 1
        wwait(k1 - 1, 0)
        gissue(k1 + 1, 0)
        gwait(k1, 1)
        wissue(k1, 1)
        k2 = 2 * p
        wwait(k2 - 1, 1)
        gissue(k2 + 1, 1)
        gwait(k2, 0)
        wissue(k2, 0)
        return 0

    lax.fori_loop(1, n_chunks // 2, pair, 0)
    gwait(n_chunks - 1, 1)
    wissue(n_chunks - 1, 1)
    wwait(n_chunks - 2, 0)
    wwait(n_chunks - 1, 1)


_ef_kernel = pl.kernel(
    _ef_body,
    out_type=(
        jax.ShapeDtypeStruct((EP, D_OUT), jnp.float32),
        jax.ShapeDtypeStruct((EP, D_OUT), jnp.float32),
    ),
    mesh=_MESH,
    scratch_types=[
        pltpu.VMEM((_E_PER_W,), jnp.int32),
        pltpu.VMEM((_E_PER_W,), jnp.int32),
        pltpu.VMEM((128, D_OUT), jnp.float32),
        pltpu.VMEM((128, D_OUT), jnp.float32),
        pltpu.VMEM((128, D_OUT), jnp.float32),
        pltpu.VMEM((128, D_OUT), jnp.float32),
        pltpu.SemaphoreType.DMA,
        pltpu.SemaphoreType.DMA,
        pltpu.SemaphoreType.DMA,
        pltpu.SemaphoreType.DMA,
    ],
    name="ef_gather",
)


_BN = 400
_NB = N // _BN


def _dinv_body(deg_ref, dinv_ref):
    deg = deg_ref[0, :, 0:1] + deg_ref[1, :, 0:1] + 1.0
    dinv_ref[...] = jnp.where(deg > 0, lax.rsqrt(deg), 0.0)


def _dinv_kernel(deg):
    return pl.pallas_call(
        _dinv_body,
        grid=(_NB,),
        in_specs=[pl.BlockSpec((2, _BN, HC), lambda i: (0, i, 0))],
        out_specs=pl.BlockSpec((_BN, 1), lambda i: (i, 0)),
        out_shape=jax.ShapeDtypeStruct((N, 1), jnp.float32),
        name="dinv",
    )(deg)


def _mm1_body(x_ref, w_ref, d_ref, out_ref):
    out_ref[...] = d_ref[...] * jnp.dot(x_ref[...], w_ref[...],
                                        preferred_element_type=jnp.float32)


def _mm1(x, W1, dinv):
    return pl.pallas_call(
        _mm1_body,
        grid=(_NB, NC),
        in_specs=[
            pl.BlockSpec((_BN, D), lambda i, c: (i, 0)),
            pl.BlockSpec((D, D // 2), lambda i, c: (0, c)),
            pl.BlockSpec((_BN, 1), lambda i, c: (i, 0)),
        ],
        out_specs=pl.BlockSpec((_BN, D // 2), lambda i, c: (c * _NB + i, 0)),
        out_shape=jax.ShapeDtypeStruct((2 * N, D // 2), jnp.float32),
        name="mm1",
    )(x, W1, dinv)


def _mm_next_body(hw, sA_ref, sB_ref, gA_ref, gB_ref, d_ref, b_ref, w_ref,
                  out_ref):
    d = d_ref[...]
    b = b_ref[...]
    hA = jnp.maximum(d * (sA_ref[0] + gA_ref[...]) + b[:, :hw], 0.0)
    hB = jnp.maximum(d * (sB_ref[0] + gB_ref[...]) + b[:, hw:], 0.0)
    out_ref[...] = d * (
        jnp.dot(hA, w_ref[:hw, :], preferred_element_type=jnp.float32)
        + jnp.dot(hB, w_ref[hw:, :], preferred_element_type=jnp.float32)
    )


def _mm_next(s, g, dinv, b, W, h_in, h_out, split_out=True):
    hw = h_in // 2
    if split_out:
        ow = h_out // 2
        return pl.pallas_call(
            functools.partial(_mm_next_body, hw),
            grid=(_NB, NC),
            in_specs=[
                pl.BlockSpec((1, _BN, hw), lambda i, c: (0, i, 0)),
                pl.BlockSpec((1, _BN, hw), lambda i, c: (1, i, 0)),
                pl.BlockSpec((_BN, hw), lambda i, c: (i, 0)),
                pl.BlockSpec((_BN, hw), lambda i, c: (_NB + i, 0)),
                pl.BlockSpec((_BN, 1), lambda i, c: (i, 0)),
                pl.BlockSpec((1, h_in), lambda i, c: (0, 0)),
                pl.BlockSpec((h_in, ow), lambda i, c: (0, c)),
            ],
            out_specs=pl.BlockSpec((_BN, ow), lambda i, c: (c * _NB + i, 0)),
            out_shape=jax.ShapeDtypeStruct((2 * N, ow), jnp.float32),
            name="mm_next",
        )(s, s, g, g, dinv, b, W)
    return pl.pallas_call(
        functools.partial(_mm_next_body, hw),
        grid=(_NB,),
        in_specs=[
            pl.BlockSpec((1, _BN, hw), lambda i: (0, i, 0)),
            pl.BlockSpec((1, _BN, hw), lambda i: (1, i, 0)),
            pl.BlockSpec((_BN, hw), lambda i: (i, 0)),
            pl.BlockSpec((_BN, hw), lambda i: (_NB + i, 0)),
            pl.BlockSpec((_BN, 1), lambda i: (i, 0)),
            pl.BlockSpec((1, h_in), lambda i: (0, 0)),
            pl.BlockSpec((h_in, h_out), lambda i: (0, 0)),
        ],
        out_specs=pl.BlockSpec((_BN, h_out), lambda i: (i, 0)),
        out_shape=jax.ShapeDtypeStruct((N, h_out), jnp.float32),
        name="mm_next_full",
    )(s, s, g, g, dinv, b, W)


def _emb_body(s0_ref, s1_ref, g_ref, d_ref, b_ref, xout_ref, emb_ref):
    xo = d_ref[...] * (s0_ref[0] + s1_ref[0] + g_ref[...]) + b_ref[...]
    xout_ref[...] = xo
    emb_ref[...] = jnp.maximum(xo, 0.0)


def _emb_kernel(s3, g3, dinv, b3):
    return pl.pallas_call(
        _emb_body,
        grid=(_NB,),
        in_specs=[
            pl.BlockSpec((1, _BN, D_OUT), lambda i: (0, i, 0)),
            pl.BlockSpec((1, _BN, D_OUT), lambda i: (1, i, 0)),
            pl.BlockSpec((_BN, D_OUT), lambda i: (i, 0)),
            pl.BlockSpec((_BN, 1), lambda i: (i, 0)),
            pl.BlockSpec((1, D_OUT), lambda i: (0, 0)),
        ],
        out_specs=(
            pl.BlockSpec((_BN, D_OUT), lambda i: (i, 0)),
            pl.BlockSpec((_BN, D_OUT), lambda i: (i, 0)),
        ),
        out_shape=(
            jax.ShapeDtypeStruct((N, D_OUT), jnp.float32),
            jax.ShapeDtypeStruct((N, D_OUT), jnp.float32),
        ),
        name="emb",
    )(s3, s3, g3, dinv, b3)


_BE = 640
_EB = E // _BE


def _mlp_body(efs_ref, efd_ref, w_ref, m1a_ref, m1r_ref, m1b_ref, mb1_ref,
              m2_ref, mb2_ref, m3_ref, mb3_ref, out_ref):
    m = (
        jnp.dot(efs_ref[...], m1a_ref[...], preferred_element_type=jnp.float32)
        + jnp.dot(efd_ref[...], m1b_ref[...], preferred_element_type=jnp.float32)
        + w_ref[...] * m1r_ref[...]
        + mb1_ref[...]
    )
    m = jnp.maximum(m, 0.0)
    m = jnp.dot(m, m2_ref[...], preferred_element_type=jnp.float32) + mb2_ref[...]
    m = jnp.maximum(m, 0.0)
    out_ref[...] = (
        jnp.dot(m, m3_ref[...], preferred_element_type=jnp.float32) + mb3_ref[...]
    )


def _mlp_kernel(efs, efd, ew, M1a, M1r, M1b, mb1, M2, mb2, M3, mb3):
    def full(shape):
        return pl.BlockSpec(shape, lambda i: tuple(0 for _ in shape))

    return pl.pallas_call(
        _mlp_body,
        grid=(_EB,),
        in_specs=[
            pl.BlockSpec((_BE, D_OUT), lambda i: (i, 0)),
            pl.BlockSpec((_BE, D_OUT), lambda i: (i, 0)),
            pl.BlockSpec((_BE, 1), lambda i: (i, 0)),
            full((D_OUT, MLP_H)),
            full((1, MLP_H)),
            full((D_OUT, MLP_H)),
            full((1, MLP_H)),
            full((MLP_H, MLP_H)),
            full((1, MLP_H)),
            full((MLP_H, 2)),
            full((1, 2)),
        ],
        out_specs=pl.BlockSpec((_BE, 2), lambda i: (i, 0)),
        out_shape=jax.ShapeDtypeStruct((E, 2), jnp.float32),
        name="edge_mlp",
    )(efs, efd, ew, M1a, M1r, M1b, mb1, M2, mb2, M3, mb3)


def kernel(x, edge_index, edge_weight, W1, b1, W2, b2, W3, b3,
           M1, mb1, M2, mb2, M3, mb3):
    npad = EP - E
    pad_idx = (jnp.arange(npad, dtype=jnp.int32) * 97) % N
    src = jnp.concatenate([edge_index[0], pad_idx])
    dst = jnp.concatenate([edge_index[1], pad_idx])
    w = jnp.concatenate([edge_weight[:, 0], jnp.zeros((npad,), jnp.float32)])

    deg = _edge_pass_deg(w, src, dst)
    dinv = _dinv_kernel(deg)

    g1 = _mm1(x, W1, dinv)
    s1 = _edge_pass_feat(g1, w, src, dst)
    g2 = _mm_next(s1, g1, dinv, b1.reshape(1, -1), W2, D, D)
    s2 = _edge_pass_feat(g2, w, src, dst)
    g3 = _mm_next(s2, g2, dinv, b2.reshape(1, -1), W3, D, D_OUT,
                  split_out=False)
    s3 = _edge_pass_edge(g3, w, src, dst)
    x_out, emb = _emb_kernel(s3, g3, dinv, b3.reshape(1, -1))

    efs, efd = _ef_kernel(emb, src, dst)
    edge_out = _mlp_kernel(
        efs, efd, edge_weight,
        M1[:D_OUT], M1[D_OUT:D_OUT + 1], M1[D_OUT + 1:],
        mb1.reshape(1, -1), M2, mb2.reshape(1, -1), M3, mb3.reshape(1, -1),
    )
    return (x_out, edge_out)


# speedup vs baseline: 7.7689x; 1.0654x over previous
import functools

import jax
import jax.numpy as jnp
from jax import lax
from jax.experimental import pallas as pl
from jax.experimental.pallas import tpu as pltpu
from jax.experimental.pallas import tpu_sc as plsc

N = 10000
E = 160000
EP = 163840
D = 256
D_OUT = 128
MLP_H = 256

NC = 2
NS = 16
L = 16
NP = 10240

_MESH = plsc.VectorSubcoreMesh(
    core_axis_name="c", subcore_axis_name="s", num_cores=NC, num_subcores=NS
)


HC = 128


def _make_edge_pass(mode):
    per_core = (mode == "feat")
    e_per_tile = EP // NS if per_core else EP // (NC * NS)
    C = 64
    n_chunks = e_per_tile // C
    rows_per_tile = NP // NS
    zrows = 64
    gather = (mode != "deg")

    def body(*refs):
        if gather:
            table, scale, src, dst, out = refs[:5]
            scr = refs[5:]
        else:
            scale, src, dst, out = refs[:4]
            scr = refs[4:]
        (src_v, dst_v, scale_v, src_c0, src_c1, dst_c0, dst_c1, rows0, rows1,
         accum, gsem0, gsem1, ssem0, ssem1, zsem) = scr
        src_c = (src_c0, src_c1)
        dst_c = (dst_c0, dst_c1)
        rows = (rows0, rows1)
        gsem = (gsem0, gsem1)
        ssem = (ssem0, ssem1)
        cid = lax.axis_index("c")
        sid = lax.axis_index("s")
        if per_core:
            ebase = sid * e_per_tile
        else:
            ebase = (cid * NS + sid) * e_per_tile

        zero = jnp.zeros((L,), jnp.float32)

        def zbody(r, _):
            for j in range(HC // L):
                rows0[r, pl.ds(j * L, L)] = zero
            return 0

        lax.fori_loop(0, zrows, zbody, 0)
        r0 = sid * rows_per_tile
        for t in range(rows_per_tile // zrows):
            pltpu.make_async_copy(
                rows0, accum.at[pl.ds(r0 + t * zrows, zrows)], zsem).start()

        if gather:
            pltpu.sync_copy(src.at[pl.ds(ebase, e_per_tile)], src_v)
        pltpu.sync_copy(dst.at[pl.ds(ebase, e_per_tile)], dst_v)
        pltpu.sync_copy(scale.at[pl.ds(ebase, e_per_tile)], scale_v)

        for t in range(rows_per_tile // zrows):
            pltpu.make_async_copy(
                rows0, accum.at[pl.ds(r0 + t * zrows, zrows)], zsem).wait()
        plsc.subcore_barrier()

        off = cid * N

        def prep(k, b):
            cb = k * C
            for j in range(C // L):
                sl = pl.ds(cb + j * L, L)
                if gather:
                    idx = src_v[sl]
                    src_c[b][pl.ds(j * L, L)] = idx + off if per_core else idx
                dst_c[b][pl.ds(j * L, L)] = dst_v[sl]

        def gissue(b):
            pltpu.make_async_copy(table.at[src_c[b]], rows[b], gsem[b]).start()

        def gwait(b):
            pltpu.make_async_copy(table.at[src_c[b]], rows[b], gsem[b]).wait()

        def sissue(b):
            pltpu.async_copy(rows[b], accum.at[dst_c[b]], ssem[b], add=True)

        def swait(b):
            pltpu.make_async_copy(rows[b], accum.at[dst_c[b]], ssem[b]).wait()

        def scale_chunk(k, b):
            cb = k * C

            def ebody(m, _):
                sv = scale_v[pl.ds(cb + m * L, L)]
                for e in range(L):
                    sc = sv[e]
                    for j in range(HC // L):
                        sl = pl.ds(j * L, L)
                        if gather:
                            rows[b][m * L + e, sl] = rows[b][m * L + e, sl] * sc
                        else:
                            rows[b][m * L + e, sl] = jnp.full((L,), sc)
                return 0

            lax.fori_loop(0, C // L, ebody, 0)

        if gather:
            prep(0, 0)
            gissue(0)
            prep(1, 1)
            gissue(1)
            gwait(0)
            scale_chunk(0, 0)
            sissue(0)

            def pair(p, _):
                k1 = 2 * p - 1
                swait(0)
                prep(k1 + 1, 0)
                gissue(0)
                gwait(1)
                scale_chunk(k1, 1)
                sissue(1)
                k2 = 2 * p
                swait(1)
                prep(k2 + 1, 1)
                gissue(1)
                gwait(0)
                scale_chunk(k2, 0)
                sissue(0)
                return 0

            lax.fori_loop(1, n_chunks // 2, pair, 0)
            gwait(1)
            scale_chunk(n_chunks - 1, 1)
            sissue(1)
            swait(0)
            swait(1)
        else:
            prep(0, 0)
            scale_chunk(0, 0)
            sissue(0)
            prep(1, 1)
            scale_chunk(1, 1)
            sissue(1)

            def pair(p, _):
                k1 = 2 * p
                swait(0)
                prep(k1, 0)
                scale_chunk(k1, 0)
                sissue(0)
                k2 = 2 * p + 1
                swait(1)
                prep(k2, 1)
                scale_chunk(k2, 1)
                sissue(1)
                return 0

            lax.fori_loop(1, n_chunks // 2, pair, 0)
            swait(0)
            swait(1)

        plsc.subcore_barrier()
        pltpu.sync_copy(
            accum.at[pl.ds(r0, rows_per_tile)],
            out.at[cid, pl.ds(r0, rows_per_tile)],
        )

    return pl.kernel(
        body,
        out_type=jax.ShapeDtypeStruct((NC, NP, HC), jnp.float32),
        mesh=_MESH,
        scratch_types=[
            pltpu.VMEM((e_per_tile,), jnp.int32),
            pltpu.VMEM((e_per_tile,), jnp.int32),
            pltpu.VMEM((e_per_tile,), jnp.float32),
            pltpu.VMEM((C,), jnp.int32),
            pltpu.VMEM((C,), jnp.int32),
            pltpu.VMEM((C,), jnp.int32),
            pltpu.VMEM((C,), jnp.int32),
            pltpu.VMEM((C, HC), jnp.float32),
            pltpu.VMEM((C, HC), jnp.float32),
            pltpu.VMEM_SHARED((NP, HC), jnp.float32),
            pltpu.SemaphoreType.DMA,
            pltpu.SemaphoreType.DMA,
            pltpu.SemaphoreType.DMA,
            pltpu.SemaphoreType.DMA,
            pltpu.SemaphoreType.DMA,
        ],
        name=f"edge_pass_{mode}",
    )


_edge_pass_feat = _make_edge_pass("feat")
_edge_pass_edge = _make_edge_pass("edge")
_edge_pass_deg = _make_edge_pass("deg")


_E_PER_W = EP // (NC * NS)


def _ef_body(emb, src, dst, out_s, out_d, src_v, dst_v,
             rs0, rs1, rd0, rd1, gsem0, gsem1, wsem0, wsem1):
    C = 128
    n_chunks = _E_PER_W // C
    wid = lax.axis_index("c") * NS + lax.axis_index("s")
    base = wid * _E_PER_W
    pltpu.sync_copy(src.at[pl.ds(base, _E_PER_W)], src_v)
    pltpu.sync_copy(dst.at[pl.ds(base, _E_PER_W)], dst_v)
    rs = (rs0, rs1)
    rd = (rd0, rd1)
    gsem = (gsem0, gsem1)
    wsem = (wsem0, wsem1)

    def gissue(k, b):
        cb = k * C
        pltpu.make_async_copy(
            emb.at[src_v.at[pl.ds(cb, C)]], rs[b], gsem[b]).start()
        pltpu.make_async_copy(
            emb.at[dst_v.at[pl.ds(cb, C)]], rd[b], gsem[b]).start()

    def gwait(k, b):
        cb = k * C
        pltpu.make_async_copy(
            emb.at[src_v.at[pl.ds(cb, C)]], rs[b], gsem[b]).wait()
        pltpu.make_async_copy(
            emb.at[dst_v.at[pl.ds(cb, C)]], rd[b], gsem[b]).wait()

    def wissue(k, b):
        cb = k * C
        pltpu.make_async_copy(rs[b], out_s.at[pl.ds(base + cb, C)],
                              wsem[b]).start()
        pltpu.make_async_copy(rd[b], out_d.at[pl.ds(base + cb, C)],
                              wsem[b]).start()

    def wwait(k, b):
        cb = k * C
        pltpu.make_async_copy(rs[b], out_s.at[pl.ds(base + cb, C)],
                              wsem[b]).wait()
        pltpu.make_async_copy(rd[b], out_d.at[pl.ds(base + cb, C)],
                              wsem[b]).wait()

    gissue(0, 0)
    gissue(1, 1)
    gwait(0, 0)
    wissue(0, 0)

    def pair(p, _):
        k1 = 2 * p - 1
        wwait(k1 - 1, 0)
        gissue(k1 + 1, 0)
        gwait(k1, 1)
        wissue(k1, 1)
        k2 = 2 * p
        wwait(k2 - 1, 1)
        gissue(k2 + 1, 1)
        gwait(k2, 0)
        wissue(k2, 0)
        return 0

    lax.fori_loop(1, n_chunks // 2, pair, 0)
    gwait(n_chunks - 1, 1)
    wissue(n_chunks - 1, 1)
    wwait(n_chunks - 2, 0)
    wwait(n_chunks - 1, 1)


_ef_kernel = pl.kernel(
    _ef_body,
    out_type=(
        jax.ShapeDtypeStruct((EP, D_OUT), jnp.float32),
        jax.ShapeDtypeStruct((EP, D_OUT), jnp.float32),
    ),
    mesh=_MESH,
    scratch_types=[
        pltpu.VMEM((_E_PER_W,), jnp.int32),
        pltpu.VMEM((_E_PER_W,), jnp.int32),
        pltpu.VMEM((128, D_OUT), jnp.float32),
        pltpu.VMEM((128, D_OUT), jnp.float32),
        pltpu.VMEM((128, D_OUT), jnp.float32),
        pltpu.VMEM((128, D_OUT), jnp.float32),
        pltpu.SemaphoreType.DMA,
        pltpu.SemaphoreType.DMA,
        pltpu.SemaphoreType.DMA,
        pltpu.SemaphoreType.DMA,
    ],
    name="ef_gather",
)


_BN = 1000
_NB = N // _BN


def _dinv_body(deg_ref, raw_ref, dinv_ref, g_ref):
    deg = deg_ref[0, :, 0:1] + deg_ref[1, :, 0:1] + 1.0
    dinv = jnp.where(deg > 0, lax.rsqrt(deg), 0.0)

    @pl.when(pl.program_id(1) == 0)
    def _():
        dinv_ref[...] = dinv

    g_ref[...] = dinv * raw_ref[...]


def _dinv_kernel(deg, raw):
    return pl.pallas_call(
        _dinv_body,
        grid=(_NB, NC),
        in_specs=[
            pl.BlockSpec((2, _BN, HC), lambda i, c: (0, i, 0)),
            pl.BlockSpec((_BN, D // 2), lambda i, c: (c * _NB + i, 0)),
        ],
        out_specs=(
            pl.BlockSpec((_BN, 1), lambda i, c: (i, 0)),
            pl.BlockSpec((_BN, D // 2), lambda i, c: (c * _NB + i, 0)),
        ),
        out_shape=(
            jax.ShapeDtypeStruct((N, 1), jnp.float32),
            jax.ShapeDtypeStruct((2 * N, D // 2), jnp.float32),
        ),
        name="dinv_scale",
    )(deg, raw)


def _mm1_body(x_ref, w_ref, out_ref):
    out_ref[...] = jnp.dot(x_ref[...], w_ref[...],
                           preferred_element_type=jnp.float32)


def _mm1(x, W1):
    return pl.pallas_call(
        _mm1_body,
        grid=(_NB, NC),
        in_specs=[
            pl.BlockSpec((_BN, D), lambda i, c: (i, 0)),
            pl.BlockSpec((D, D // 2), lambda i, c: (0, c)),
        ],
        out_specs=pl.BlockSpec((_BN, D // 2), lambda i, c: (c * _NB + i, 0)),
        out_shape=jax.ShapeDtypeStruct((2 * N, D // 2), jnp.float32),
        name="mm1",
    )(x, W1)


def _mm_next_body(hw, sA_ref, sB_ref, gA_ref, gB_ref, d_ref, b_ref, w_ref,
                  out_ref):
    d = d_ref[...]
    b = b_ref[...]
    hA = jnp.maximum(d * (sA_ref[0] + gA_ref[...]) + b[:, :hw], 0.0)
    hB = jnp.maximum(d * (sB_ref[0] + gB_ref[...]) + b[:, hw:], 0.0)
    out_ref[...] = d * (
        jnp.dot(hA, w_ref[:hw, :], preferred_element_type=jnp.float32)
        + jnp.dot(hB, w_ref[hw:, :], preferred_element_type=jnp.float32)
    )


def _mm_next(s, g, dinv, b, W, h_in, h_out, split_out=True):
    hw = h_in // 2
    if split_out:
        ow = h_out // 2
        return pl.pallas_call(
            functools.partial(_mm_next_body, hw),
            grid=(_NB, NC),
            in_specs=[
                pl.BlockSpec((1, _BN, hw), lambda i, c: (0, i, 0)),
                pl.BlockSpec((1, _BN, hw), lambda i, c: (1, i, 0)),
                pl.BlockSpec((_BN, hw), lambda i, c: (i, 0)),
                pl.BlockSpec((_BN, hw), lambda i, c: (_NB + i, 0)),
                pl.BlockSpec((_BN, 1), lambda i, c: (i, 0)),
                pl.BlockSpec((1, h_in), lambda i, c: (0, 0)),
                pl.BlockSpec((h_in, ow), lambda i, c: (0, c)),
            ],
            out_specs=pl.BlockSpec((_BN, ow), lambda i, c: (c * _NB + i, 0)),
            out_shape=jax.ShapeDtypeStruct((2 * N, ow), jnp.float32),
            name="mm_next",
        )(s, s, g, g, dinv, b, W)
    return pl.pallas_call(
        functools.partial(_mm_next_body, hw),
        grid=(_NB,),
        in_specs=[
            pl.BlockSpec((1, _BN, hw), lambda i: (0, i, 0)),
            pl.BlockSpec((1, _BN, hw), lambda i: (1, i, 0)),
            pl.BlockSpec((_BN, hw), lambda i: (i, 0)),
            pl.BlockSpec((_BN, hw), lambda i: (_NB + i, 0)),
            pl.BlockSpec((_BN, 1), lambda i: (i, 0)),
            pl.BlockSpec((1, h_in), lambda i: (0, 0)),
            pl.BlockSpec((h_in, h_out), lambda i: (0, 0)),
        ],
        out_specs=pl.BlockSpec((_BN, h_out), lambda i: (i, 0)),
        out_shape=jax.ShapeDtypeStruct((N, h_out), jnp.float32),
        name="mm_next_full",
    )(s, s, g, g, dinv, b, W)


def _emb_body(s0_ref, s1_ref, g_ref, d_ref, b_ref, xout_ref, emb_ref):
    xo = d_ref[...] * (s0_ref[0] + s1_ref[0] + g_ref[...]) + b_ref[...]
    xout_ref[...] = xo
    emb_ref[...] = jnp.maximum(xo, 0.0)


def _emb_kernel(s3, g3, dinv, b3):
    return pl.pallas_call(
        _emb_body,
        grid=(_NB,),
        in_specs=[
            pl.BlockSpec((1, _BN, D_OUT), lambda i: (0, i, 0)),
            pl.BlockSpec((1, _BN, D_OUT), lambda i: (1, i, 0)),
            pl.BlockSpec((_BN, D_OUT), lambda i: (i, 0)),
            pl.BlockSpec((_BN, 1), lambda i: (i, 0)),
            pl.BlockSpec((1, D_OUT), lambda i: (0, 0)),
        ],
        out_specs=(
            pl.BlockSpec((_BN, D_OUT), lambda i: (i, 0)),
            pl.BlockSpec((_BN, D_OUT), lambda i: (i, 0)),
        ),
        out_shape=(
            jax.ShapeDtypeStruct((N, D_OUT), jnp.float32),
            jax.ShapeDtypeStruct((N, D_OUT), jnp.float32),
        ),
        name="emb",
    )(s3, s3, g3, dinv, b3)


_BE = 640
_EB = E // _BE


def _mlp_body(efs_ref, efd_ref, w_ref, m1a_ref, m1r_ref, m1b_ref, mb1_ref,
              m2_ref, mb2_ref, m3_ref, mb3_ref, out_ref):
    bf = jnp.bfloat16
    m = (
        jnp.dot(efs_ref[...].astype(bf), m1a_ref[...],
                preferred_element_type=jnp.float32)
        + jnp.dot(efd_ref[...].astype(bf), m1b_ref[...],
                  preferred_element_type=jnp.float32)
        + w_ref[...] * m1r_ref[...]
        + mb1_ref[...]
    )
    m = jnp.maximum(m, 0.0)
    m = jnp.dot(m.astype(bf), m2_ref[...],
                preferred_element_type=jnp.float32) + mb2_ref[...]
    m = jnp.maximum(m, 0.0)
    out_ref[...] = (
        jnp.dot(m.astype(bf), m3_ref[...],
                preferred_element_type=jnp.float32) + mb3_ref[...]
    )


def _mlp_kernel(efs, efd, ew, M1a, M1r, M1b, mb1, M2, mb2, M3, mb3):
    def full(shape):
        return pl.BlockSpec(shape, lambda i: tuple(0 for _ in shape))

    return pl.pallas_call(
        _mlp_body,
        grid=(_EB,),
        in_specs=[
            pl.BlockSpec((_BE, D_OUT), lambda i: (i, 0)),
            pl.BlockSpec((_BE, D_OUT), lambda i: (i, 0)),
            pl.BlockSpec((_BE, 1), lambda i: (i, 0)),
            full((D_OUT, MLP_H)),
            full((1, MLP_H)),
            full((D_OUT, MLP_H)),
            full((1, MLP_H)),
            full((MLP_H, MLP_H)),
            full((1, MLP_H)),
            full((MLP_H, 2)),
            full((1, 2)),
        ],
        out_specs=pl.BlockSpec((_BE, 2), lambda i: (i, 0)),
        out_shape=jax.ShapeDtypeStruct((E, 2), jnp.float32),
        name="edge_mlp",
    )(efs, efd, ew, M1a, M1r, M1b, mb1, M2, mb2, M3, mb3)


def kernel(x, edge_index, edge_weight, W1, b1, W2, b2, W3, b3,
           M1, mb1, M2, mb2, M3, mb3):
    npad = EP - E
    pad_idx = (jnp.arange(npad, dtype=jnp.int32) * 97) % N
    src = jnp.concatenate([edge_index[0], pad_idx])
    dst = jnp.concatenate([edge_index[1], pad_idx])
    w = jnp.concatenate([edge_weight[:, 0], jnp.zeros((npad,), jnp.float32)])

    deg = _edge_pass_deg(w, src, dst)
    g1raw = _mm1(x, W1)
    dinv, g1 = _dinv_kernel(deg, g1raw)
    s1 = _edge_pass_feat(g1, w, src, dst)
    g2 = _mm_next(s1, g1, dinv, b1.reshape(1, -1), W2, D, D)
    s2 = _edge_pass_feat(g2, w, src, dst)
    g3 = _mm_next(s2, g2, dinv, b2.reshape(1, -1), W3, D, D_OUT,
                  split_out=False)
    s3 = _edge_pass_edge(g3, w, src, dst)
    x_out, emb = _emb_kernel(s3, g3, dinv, b3.reshape(1, -1))

    efs, efd = _ef_kernel(emb, src, dst)
    bf = jnp.bfloat16
    edge_out = _mlp_kernel(
        efs, efd, edge_weight,
        M1[:D_OUT].astype(bf), M1[D_OUT:D_OUT + 1], M1[D_OUT + 1:].astype(bf),
        mb1.reshape(1, -1), M2.astype(bf), mb2.reshape(1, -1),
        M3.astype(bf), mb3.reshape(1, -1),
    )
    return (x_out, edge_out)


# speedup vs baseline: 7.7851x; 1.0021x over previous
import functools

import jax
import jax.numpy as jnp
from jax import lax
from jax.experimental import pallas as pl
from jax.experimental.pallas import tpu as pltpu
from jax.experimental.pallas import tpu_sc as plsc

N = 10000
E = 160000
EP = 163840
D = 256
D_OUT = 128
MLP_H = 256

NC = 2
NS = 16
L = 16
NP = 10240

_MESH = plsc.VectorSubcoreMesh(
    core_axis_name="c", subcore_axis_name="s", num_cores=NC, num_subcores=NS
)


HC = 128


def _make_edge_pass(mode):
    per_core = (mode == "feat")
    e_per_tile = EP // NS if per_core else EP // (NC * NS)
    C = 64
    n_chunks = e_per_tile // C
    rows_per_tile = NP // NS
    zrows = 64
    gather = (mode != "deg")

    def body(*refs):
        if gather:
            table, scale, src, dst, out = refs[:5]
            scr = refs[5:]
        else:
            scale, src, dst, out = refs[:4]
            scr = refs[4:]
        (src_v, dst_v, scale_v, src_c0, src_c1, dst_c0, dst_c1, rows0, rows1,
         accum, gsem0, gsem1, ssem0, ssem1, zsem) = scr
        src_c = (src_c0, src_c1)
        dst_c = (dst_c0, dst_c1)
        rows = (rows0, rows1)
        gsem = (gsem0, gsem1)
        ssem = (ssem0, ssem1)
        cid = lax.axis_index("c")
        sid = lax.axis_index("s")
        if per_core:
            ebase = sid * e_per_tile
        else:
            ebase = (cid * NS + sid) * e_per_tile

        zero = jnp.zeros((L,), jnp.float32)

        def zbody(r, _):
            for j in range(HC // L):
                rows0[r, pl.ds(j * L, L)] = zero
            return 0

        lax.fori_loop(0, zrows, zbody, 0)
        r0 = sid * rows_per_tile
        for t in range(rows_per_tile // zrows):
            pltpu.make_async_copy(
                rows0, accum.at[pl.ds(r0 + t * zrows, zrows)], zsem).start()

        if gather:
            pltpu.sync_copy(src.at[pl.ds(ebase, e_per_tile)], src_v)
        pltpu.sync_copy(dst.at[pl.ds(ebase, e_per_tile)], dst_v)
        pltpu.sync_copy(scale.at[pl.ds(ebase, e_per_tile)], scale_v)

        for t in range(rows_per_tile // zrows):
            pltpu.make_async_copy(
                rows0, accum.at[pl.ds(r0 + t * zrows, zrows)], zsem).wait()
        plsc.subcore_barrier()

        off = cid * N

        def prep(k, b):
            cb = k * C
            for j in range(C // L):
                sl = pl.ds(cb + j * L, L)
                if gather:
                    idx = src_v[sl]
                    src_c[b][pl.ds(j * L, L)] = idx + off if per_core else idx
                dst_c[b][pl.ds(j * L, L)] = dst_v[sl]

        def gissue(b):
            pltpu.make_async_copy(table.at[src_c[b]], rows[b], gsem[b]).start()

        def gwait(b):
            pltpu.make_async_copy(table.at[src_c[b]], rows[b], gsem[b]).wait()

        def sissue(b):
            pltpu.async_copy(rows[b], accum.at[dst_c[b]], ssem[b], add=True)

        def swait(b):
            pltpu.make_async_copy(rows[b], accum.at[dst_c[b]], ssem[b]).wait()

        def scale_chunk(k, b):
            cb = k * C

            def ebody(m, _):
                sv = scale_v[pl.ds(cb + m * L, L)]
                for e in range(L):
                    sc = sv[e]
                    for j in range(HC // L):
                        sl = pl.ds(j * L, L)
                        if gather:
                            rows[b][m * L + e, sl] = rows[b][m * L + e, sl] * sc
                        else:
                            rows[b][m * L + e, sl] = jnp.full((L,), sc)
                return 0

            lax.fori_loop(0, C // L, ebody, 0)

        if gather:
            prep(0, 0)
            gissue(0)
            prep(1, 1)
            gissue(1)
            gwait(0)
            scale_chunk(0, 0)
            sissue(0)

            def pair(p, _):
                k1 = 2 * p - 1
                swait(0)
                prep(k1 + 1, 0)
                gissue(0)
                gwait(1)
                scale_chunk(k1, 1)
                sissue(1)
                k2 = 2 * p
                swait(1)
                prep(k2 + 1, 1)
                gissue(1)
                gwait(0)
                scale_chunk(k2, 0)
                sissue(0)
                return 0

            lax.fori_loop(1, n_chunks // 2, pair, 0)
            gwait(1)
            scale_chunk(n_chunks - 1, 1)
            sissue(1)
            swait(0)
            swait(1)
        else:
            prep(0, 0)
            scale_chunk(0, 0)
            sissue(0)
            prep(1, 1)
            scale_chunk(1, 1)
            sissue(1)

            def pair(p, _):
                k1 = 2 * p
                swait(0)
                prep(k1, 0)
                scale_chunk(k1, 0)
                sissue(0)
                k2 = 2 * p + 1
                swait(1)
                prep(k2, 1)
                scale_chunk(k2, 1)
                sissue(1)
                return 0

            lax.fori_loop(1, n_chunks // 2, pair, 0)
            swait(0)
            swait(1)

        plsc.subcore_barrier()
        pltpu.sync_copy(
            accum.at[pl.ds(r0, rows_per_tile)],
            out.at[cid, pl.ds(r0, rows_per_tile)],
        )

    return pl.kernel(
        body,
        out_type=jax.ShapeDtypeStruct((NC, NP, HC), jnp.float32),
        mesh=_MESH,
        scratch_types=[
            pltpu.VMEM((e_per_tile,), jnp.int32),
            pltpu.VMEM((e_per_tile,), jnp.int32),
            pltpu.VMEM((e_per_tile,), jnp.float32),
            pltpu.VMEM((C,), jnp.int32),
            pltpu.VMEM((C,), jnp.int32),
            pltpu.VMEM((C,), jnp.int32),
            pltpu.VMEM((C,), jnp.int32),
            pltpu.VMEM((C, HC), jnp.float32),
            pltpu.VMEM((C, HC), jnp.float32),
            pltpu.VMEM_SHARED((NP, HC), jnp.float32),
            pltpu.SemaphoreType.DMA,
            pltpu.SemaphoreType.DMA,
            pltpu.SemaphoreType.DMA,
            pltpu.SemaphoreType.DMA,
            pltpu.SemaphoreType.DMA,
        ],
        name=f"edge_pass_{mode}",
    )


_edge_pass_feat = _make_edge_pass("feat")
_edge_pass_edge = _make_edge_pass("edge")
_edge_pass_deg = _make_edge_pass("deg")


_E_PER_W = EP // (NC * NS)


def _ef_body(emb, src, dst, out_s, out_d, src_v, dst_v,
             rs0, rs1, rd0, rd1, gsem0, gsem1, wsem0, wsem1):
    C = 128
    n_chunks = _E_PER_W // C
    wid = lax.axis_index("c") * NS + lax.axis_index("s")
    base = wid * _E_PER_W
    pltpu.sync_copy(src.at[pl.ds(base, _E_PER_W)], src_v)
    pltpu.sync_copy(dst.at[pl.ds(base, _E_PER_W)], dst_v)
    rs = (rs0, rs1)
    rd = (rd0, rd1)
    gsem = (gsem0, gsem1)
    wsem = (wsem0, wsem1)

    def gissue(k, b):
        cb = k * C
        pltpu.make_async_copy(
            emb.at[src_v.at[pl.ds(cb, C)]], rs[b], gsem[b]).start()
        pltpu.make_async_copy(
            emb.at[dst_v.at[pl.ds(cb, C)]], rd[b], gsem[b]).start()

    def gwait(k, b):
        cb = k * C
        pltpu.make_async_copy(
            emb.at[src_v.at[pl.ds(cb, C)]], rs[b], gsem[b]).wait()
        pltpu.make_async_copy(
            emb.at[dst_v.at[pl.ds(cb, C)]], rd[b], gsem[b]).wait()

    def wissue(k, b):
        cb = k * C
        pltpu.make_async_copy(rs[b], out_s.at[pl.ds(base + cb, C)],
                              wsem[b]).start()
        pltpu.make_async_copy(rd[b], out_d.at[pl.ds(base + cb, C)],
                              wsem[b]).start()

    def wwait(k, b):
        cb = k * C
        pltpu.make_async_copy(rs[b], out_s.at[pl.ds(base + cb, C)],
                              wsem[b]).wait()
        pltpu.make_async_copy(rd[b], out_d.at[pl.ds(base + cb, C)],
                              wsem[b]).wait()

    gissue(0, 0)
    gissue(1, 1)
    gwait(0, 0)
    wissue(0, 0)

    def pair(p, _):
        k1 = 2 * p - 1
        wwait(k1 - 1, 0)
        gissue(k1 + 1, 0)
        gwait(k1, 1)
        wissue(k1, 1)
        k2 = 2 * p
        wwait(k2 - 1, 1)
        gissue(k2 + 1, 1)
        gwait(k2, 0)
        wissue(k2, 0)
        return 0

    lax.fori_loop(1, n_chunks // 2, pair, 0)
    gwait(n_chunks - 1, 1)
    wissue(n_chunks - 1, 1)
    wwait(n_chunks - 2, 0)
    wwait(n_chunks - 1, 1)


_ef_kernel = pl.kernel(
    _ef_body,
    out_type=(
        jax.ShapeDtypeStruct((EP, D_OUT), jnp.float32),
        jax.ShapeDtypeStruct((EP, D_OUT), jnp.float32),
    ),
    mesh=_MESH,
    scratch_types=[
        pltpu.VMEM((_E_PER_W,), jnp.int32),
        pltpu.VMEM((_E_PER_W,), jnp.int32),
        pltpu.VMEM((128, D_OUT), jnp.float32),
        pltpu.VMEM((128, D_OUT), jnp.float32),
        pltpu.VMEM((128, D_OUT), jnp.float32),
        pltpu.VMEM((128, D_OUT), jnp.float32),
        pltpu.SemaphoreType.DMA,
        pltpu.SemaphoreType.DMA,
        pltpu.SemaphoreType.DMA,
        pltpu.SemaphoreType.DMA,
    ],
    name="ef_gather",
)


_BN = 1000
_NB = N // _BN


def _dinv_body(deg_ref, raw_ref, dinv_ref, g_ref):
    deg = deg_ref[0, :, 0:1] + deg_ref[1, :, 0:1] + 1.0
    dinv = jnp.where(deg > 0, lax.rsqrt(deg), 0.0)
    dinv_ref[...] = dinv
    g_ref[...] = dinv * raw_ref[...]


def _dinv_kernel(deg, raw):
    return pl.pallas_call(
        _dinv_body,
        grid=(_NB, NC),
        in_specs=[
            pl.BlockSpec((2, _BN, HC), lambda i, c: (0, i, 0)),
            pl.BlockSpec((_BN, D // 2), lambda i, c: (c * _NB + i, 0)),
        ],
        out_specs=(
            pl.BlockSpec((_BN, 1), lambda i, c: (i, 0)),
            pl.BlockSpec((_BN, D // 2), lambda i, c: (c * _NB + i, 0)),
        ),
        out_shape=(
            jax.ShapeDtypeStruct((N, 1), jnp.float32),
            jax.ShapeDtypeStruct((2 * N, D // 2), jnp.float32),
        ),
        name="dinv_scale",
    )(deg, raw)


def _mm1_body(x_ref, w_ref, out_ref):
    out_ref[...] = jnp.dot(x_ref[...], w_ref[...],
                           preferred_element_type=jnp.float32)


def _mm1(x, W1):
    return pl.pallas_call(
        _mm1_body,
        grid=(_NB, NC),
        in_specs=[
            pl.BlockSpec((_BN, D), lambda i, c: (i, 0)),
            pl.BlockSpec((D, D // 2), lambda i, c: (0, c)),
        ],
        out_specs=pl.BlockSpec((_BN, D // 2), lambda i, c: (c * _NB + i, 0)),
        out_shape=jax.ShapeDtypeStruct((2 * N, D // 2), jnp.float32),
        name="mm1",
    )(x, W1)


def _mm_next_body(hw, sA_ref, sB_ref, gA_ref, gB_ref, d_ref, b_ref, w_ref,
                  out_ref):
    d = d_ref[...]
    b = b_ref[...]
    hA = jnp.maximum(d * (sA_ref[0] + gA_ref[...]) + b[:, :hw], 0.0)
    hB = jnp.maximum(d * (sB_ref[0] + gB_ref[...]) + b[:, hw:], 0.0)
    out_ref[...] = d * (
        jnp.dot(hA, w_ref[:hw, :], preferred_element_type=jnp.float32)
        + jnp.dot(hB, w_ref[hw:, :], preferred_element_type=jnp.float32)
    )


def _mm_next(s, g, dinv, b, W, h_in, h_out, split_out=True):
    hw = h_in // 2
    if split_out:
        ow = h_out // 2
        return pl.pallas_call(
            functools.partial(_mm_next_body, hw),
            grid=(_NB, NC),
            in_specs=[
                pl.BlockSpec((1, _BN, hw), lambda i, c: (0, i, 0)),
                pl.BlockSpec((1, _BN, hw), lambda i, c: (1, i, 0)),
                pl.BlockSpec((_BN, hw), lambda i, c: (i, 0)),
                pl.BlockSpec((_BN, hw), lambda i, c: (_NB + i, 0)),
                pl.BlockSpec((_BN, 1), lambda i, c: (i, 0)),
                pl.BlockSpec((1, h_in), lambda i, c: (0, 0)),
                pl.BlockSpec((h_in, ow), lambda i, c: (0, c)),
            ],
            out_specs=pl.BlockSpec((_BN, ow), lambda i, c: (c * _NB + i, 0)),
            out_shape=jax.ShapeDtypeStruct((2 * N, ow), jnp.float32),
            name="mm_next",
        )(s, s, g, g, dinv, b, W)
    return pl.pallas_call(
        functools.partial(_mm_next_body, hw),
        grid=(_NB,),
        in_specs=[
            pl.BlockSpec((1, _BN, hw), lambda i: (0, i, 0)),
            pl.BlockSpec((1, _BN, hw), lambda i: (1, i, 0)),
            pl.BlockSpec((_BN, hw), lambda i: (i, 0)),
            pl.BlockSpec((_BN, hw), lambda i: (_NB + i, 0)),
            pl.BlockSpec((_BN, 1), lambda i: (i, 0)),
            pl.BlockSpec((1, h_in), lambda i: (0, 0)),
            pl.BlockSpec((h_in, h_out), lambda i: (0, 0)),
        ],
        out_specs=pl.BlockSpec((_BN, h_out), lambda i: (i, 0)),
        out_shape=jax.ShapeDtypeStruct((N, h_out), jnp.float32),
        name="mm_next_full",
    )(s, s, g, g, dinv, b, W)


def _emb_body(s0_ref, s1_ref, g_ref, d_ref, b_ref, xout_ref, emb_ref):
    xo = d_ref[...] * (s0_ref[0] + s1_ref[0] + g_ref[...]) + b_ref[...]
    xout_ref[...] = xo
    emb_ref[...] = jnp.maximum(xo, 0.0)


def _emb_kernel(s3, g3, dinv, b3):
    return pl.pallas_call(
        _emb_body,
        grid=(_NB,),
        in_specs=[
            pl.BlockSpec((1, _BN, D_OUT), lambda i: (0, i, 0)),
            pl.BlockSpec((1, _BN, D_OUT), lambda i: (1, i, 0)),
            pl.BlockSpec((_BN, D_OUT), lambda i: (i, 0)),
            pl.BlockSpec((_BN, 1), lambda i: (i, 0)),
            pl.BlockSpec((1, D_OUT), lambda i: (0, 0)),
        ],
        out_specs=(
            pl.BlockSpec((_BN, D_OUT), lambda i: (i, 0)),
            pl.BlockSpec((_BN, D_OUT), lambda i: (i, 0)),
        ),
        out_shape=(
            jax.ShapeDtypeStruct((N, D_OUT), jnp.float32),
            jax.ShapeDtypeStruct((N, D_OUT), jnp.float32),
        ),
        name="emb",
    )(s3, s3, g3, dinv, b3)


_BE = 640
_EB = E // _BE


def _mlp_body(efs_ref, efd_ref, w_ref, m1a_ref, m1r_ref, m1b_ref, mb1_ref,
              m2_ref, mb2_ref, m3_ref, mb3_ref, out_ref):
    bf = jnp.bfloat16
    m = (
        jnp.dot(efs_ref[...].astype(bf), m1a_ref[...],
                preferred_element_type=jnp.float32)
        + jnp.dot(efd_ref[...].astype(bf), m1b_ref[...],
                  preferred_element_type=jnp.float32)
        + w_ref[...] * m1r_ref[...]
        + mb1_ref[...]
    )
    m = jnp.maximum(m, 0.0)
    m = jnp.dot(m.astype(bf), m2_ref[...],
                preferred_element_type=jnp.float32) + mb2_ref[...]
    m = jnp.maximum(m, 0.0)
    out_ref[...] = (
        jnp.dot(m.astype(bf), m3_ref[...],
                preferred_element_type=jnp.float32) + mb3_ref[...]
    )


def _mlp_kernel(efs, efd, ew, M1a, M1r, M1b, mb1, M2, mb2, M3, mb3):
    def full(shape):
        return pl.BlockSpec(shape, lambda i: tuple(0 for _ in shape))

    return pl.pallas_call(
        _mlp_body,
        grid=(_EB,),
        in_specs=[
            pl.BlockSpec((_BE, D_OUT), lambda i: (i, 0)),
            pl.BlockSpec((_BE, D_OUT), lambda i: (i, 0)),
            pl.BlockSpec((_BE, 1), lambda i: (i, 0)),
            full((D_OUT, MLP_H)),
            full((1, MLP_H)),
            full((D_OUT, MLP_H)),
            full((1, MLP_H)),
            full((MLP_H, MLP_H)),
            full((1, MLP_H)),
            full((MLP_H, 2)),
            full((1, 2)),
        ],
        out_specs=pl.BlockSpec((_BE, 2), lambda i: (i, 0)),
        out_shape=jax.ShapeDtypeStruct((E, 2), jnp.float32),
        name="edge_mlp",
    )(efs, efd, ew, M1a, M1r, M1b, mb1, M2, mb2, M3, mb3)


def kernel(x, edge_index, edge_weight, W1, b1, W2, b2, W3, b3,
           M1, mb1, M2, mb2, M3, mb3):
    npad = EP - E
    pad_idx = (jnp.arange(npad, dtype=jnp.int32) * 97) % N
    src = jnp.concatenate([edge_index[0], pad_idx])
    dst = jnp.concatenate([edge_index[1], pad_idx])
    w = jnp.concatenate([edge_weight[:, 0], jnp.zeros((npad,), jnp.float32)])

    deg = _edge_pass_deg(w, src, dst)
    g1raw = _mm1(x, W1)
    dinv, g1 = _dinv_kernel(deg, g1raw)
    s1 = _edge_pass_feat(g1, w, src, dst)
    g2 = _mm_next(s1, g1, dinv, b1.reshape(1, -1), W2, D, D)
    s2 = _edge_pass_feat(g2, w, src, dst)
    g3 = _mm_next(s2, g2, dinv, b2.reshape(1, -1), W3, D, D_OUT,
                  split_out=False)
    s3 = _edge_pass_edge(g3, w, src, dst)
    x_out, emb = _emb_kernel(s3, g3, dinv, b3.reshape(1, -1))

    efs, efd = _ef_kernel(emb, src, dst)
    bf = jnp.bfloat16
    edge_out = _mlp_kernel(
        efs, efd, edge_weight,
        M1[:D_OUT].astype(bf), M1[D_OUT:D_OUT + 1], M1[D_OUT + 1:].astype(bf),
        mb1.reshape(1, -1), M2.astype(bf), mb2.reshape(1, -1),
        M3.astype(bf), mb3.reshape(1, -1),
    )
    return (x_out, edge_out)


# speedup vs baseline: 8.3279x; 1.0697x over previous
import functools

import jax
import jax.numpy as jnp
from jax import lax
from jax.experimental import pallas as pl
from jax.experimental.pallas import tpu as pltpu
from jax.experimental.pallas import tpu_sc as plsc

N = 10000
E = 160000
EP = 163840
D = 256
D_OUT = 128
MLP_H = 256

NC = 2
NS = 16
L = 16
NP = 10240

_MESH = plsc.VectorSubcoreMesh(
    core_axis_name="c", subcore_axis_name="s", num_cores=NC, num_subcores=NS
)


HC = 128


def _make_edge_pass(mode):
    per_core = (mode == "feat")
    e_per_tile = EP // NS if per_core else EP // (NC * NS)
    C = 64
    n_chunks = e_per_tile // C
    rows_per_tile = NP // NS
    zrows = 64
    gather = (mode != "deg")

    def body(*refs):
        if gather:
            table, scale, src, dst, out = refs[:5]
            scr = refs[5:]
        else:
            scale, src, dst, out = refs[:4]
            scr = refs[4:]
        (src_v, dst_v, scale_v, src_c0, src_c1, dst_c0, dst_c1, rows0, rows1,
         accum, gsem0, gsem1, ssem0, ssem1, zsem) = scr
        src_c = (src_c0, src_c1)
        dst_c = (dst_c0, dst_c1)
        rows = (rows0, rows1)
        gsem = (gsem0, gsem1)
        ssem = (ssem0, ssem1)
        cid = lax.axis_index("c")
        sid = lax.axis_index("s")
        if per_core:
            ebase = sid * e_per_tile
        else:
            ebase = (cid * NS + sid) * e_per_tile

        zero = jnp.zeros((L,), jnp.float32)

        def zbody(r, _):
            for j in range(HC // L):
                rows0[r, pl.ds(j * L, L)] = zero
            return 0

        lax.fori_loop(0, zrows, zbody, 0)
        r0 = sid * rows_per_tile
        for t in range(rows_per_tile // zrows):
            pltpu.make_async_copy(
                rows0, accum.at[pl.ds(r0 + t * zrows, zrows)], zsem).start()

        if gather:
            pltpu.sync_copy(src.at[pl.ds(ebase, e_per_tile)], src_v)
        pltpu.sync_copy(dst.at[pl.ds(ebase, e_per_tile)], dst_v)
        pltpu.sync_copy(scale.at[pl.ds(ebase, e_per_tile)], scale_v)

        for t in range(rows_per_tile // zrows):
            pltpu.make_async_copy(
                rows0, accum.at[pl.ds(r0 + t * zrows, zrows)], zsem).wait()
        plsc.subcore_barrier()

        off = cid * N

        def prep(k, b):
            cb = k * C
            for j in range(C // L):
                sl = pl.ds(cb + j * L, L)
                if gather:
                    idx = src_v[sl]
                    src_c[b][pl.ds(j * L, L)] = idx + off if per_core else idx
                dst_c[b][pl.ds(j * L, L)] = dst_v[sl]

        def gissue(b):
            pltpu.make_async_copy(table.at[src_c[b]], rows[b], gsem[b]).start()

        def gwait(b):
            pltpu.make_async_copy(table.at[src_c[b]], rows[b], gsem[b]).wait()

        def sissue(b):
            pltpu.async_copy(rows[b], accum.at[dst_c[b]], ssem[b], add=True)

        def swait(b):
            pltpu.make_async_copy(rows[b], accum.at[dst_c[b]], ssem[b]).wait()

        def scale_chunk(k, b):
            cb = k * C

            def ebody(m, _):
                sv = scale_v[pl.ds(cb + m * L, L)]
                for e in range(L):
                    sc = sv[e]
                    for j in range(HC // L):
                        sl = pl.ds(j * L, L)
                        if gather:
                            rows[b][m * L + e, sl] = rows[b][m * L + e, sl] * sc
                        else:
                            rows[b][m * L + e, sl] = jnp.full((L,), sc)
                return 0

            lax.fori_loop(0, C // L, ebody, 0)

        if gather:
            prep(0, 0)
            gissue(0)
            prep(1, 1)
            gissue(1)
            gwait(0)
            scale_chunk(0, 0)
            sissue(0)

            def pair(p, _):
                k1 = 2 * p - 1
                swait(0)
                prep(k1 + 1, 0)
                gissue(0)
                gwait(1)
                scale_chunk(k1, 1)
                sissue(1)
                k2 = 2 * p
                swait(1)
                prep(k2 + 1, 1)
                gissue(1)
                gwait(0)
                scale_chunk(k2, 0)
                sissue(0)
                return 0

            lax.fori_loop(1, n_chunks // 2, pair, 0)
            gwait(1)
            scale_chunk(n_chunks - 1, 1)
            sissue(1)
            swait(0)
            swait(1)
        else:
            prep(0, 0)
            scale_chunk(0, 0)
            sissue(0)
            prep(1, 1)
            scale_chunk(1, 1)
            sissue(1)

            def pair(p, _):
                k1 = 2 * p
                swait(0)
                prep(k1, 0)
                scale_chunk(k1, 0)
                sissue(0)
                k2 = 2 * p + 1
                swait(1)
                prep(k2, 1)
                scale_chunk(k2, 1)
                sissue(1)
                return 0

            lax.fori_loop(1, n_chunks // 2, pair, 0)
            swait(0)
            swait(1)

        plsc.subcore_barrier()
        pltpu.sync_copy(
            accum.at[pl.ds(r0, rows_per_tile)],
            out.at[cid, pl.ds(r0, rows_per_tile)],
        )

    return pl.kernel(
        body,
        out_type=jax.ShapeDtypeStruct((NC, NP, HC), jnp.float32),
        mesh=_MESH,
        scratch_types=[
            pltpu.VMEM((e_per_tile,), jnp.int32),
            pltpu.VMEM((e_per_tile,), jnp.int32),
            pltpu.VMEM((e_per_tile,), jnp.float32),
            pltpu.VMEM((C,), jnp.int32),
            pltpu.VMEM((C,), jnp.int32),
            pltpu.VMEM((C,), jnp.int32),
            pltpu.VMEM((C,), jnp.int32),
            pltpu.VMEM((C, HC), jnp.float32),
            pltpu.VMEM((C, HC), jnp.float32),
            pltpu.VMEM_SHARED((NP, HC), jnp.float32),
            pltpu.SemaphoreType.DMA,
            pltpu.SemaphoreType.DMA,
            pltpu.SemaphoreType.DMA,
            pltpu.SemaphoreType.DMA,
            pltpu.SemaphoreType.DMA,
        ],
        name=f"edge_pass_{mode}",
    )


_edge_pass_feat = _make_edge_pass("feat")
_edge_pass_edge = _make_edge_pass("edge")
_edge_pass_deg = _make_edge_pass("deg")


_E_PER_W = EP // (NC * NS)


def _ef_body(emb, src, dst, out_s, out_d, src_v, dst_v,
             rs0, rs1, rd0, rd1, ps0, ps1, pd0, pd1,
             gsem0, gsem1, wsem0, wsem1):
    C = 80
    n_chunks = _E_PER_W // C
    wid = lax.axis_index("c") * NS + lax.axis_index("s")
    base = wid * _E_PER_W
    pltpu.sync_copy(src.at[pl.ds(base, _E_PER_W)], src_v)
    pltpu.sync_copy(dst.at[pl.ds(base, _E_PER_W)], dst_v)
    rs = (rs0, rs1)
    rd = (rd0, rd1)
    ps = (ps0, ps1)
    pd = (pd0, pd1)
    gsem = (gsem0, gsem1)
    wsem = (wsem0, wsem1)

    def pack_chunk(b):
        rnd = jnp.full((L,), 0x8000, jnp.int32)
        hi = jnp.full((L,), jnp.int32(-65536), jnp.int32)

        def ebody(e, _):
            for full, packed in ((rs[b], ps[b]), (rd[b], pd[b])):
                for t in range(D_OUT // (2 * L)):
                    a = full[e, pl.ds(2 * t * L, L)]
                    bb = full[e, pl.ds((2 * t + 1) * L, L)]
                    lo16 = lax.shift_right_logical(a + rnd, 16)
                    hi16 = (bb + rnd) & hi
                    packed[e, pl.ds(t * L, L)] = hi16 | lo16
            return 0

        lax.fori_loop(0, C, ebody, 0)

    def gissue(k, b):
        cb = k * C
        pltpu.make_async_copy(
            emb.at[src_v.at[pl.ds(cb, C)]], rs[b], gsem[b]).start()
        pltpu.make_async_copy(
            emb.at[dst_v.at[pl.ds(cb, C)]], rd[b], gsem[b]).start()

    def gwait(k, b):
        cb = k * C
        pltpu.make_async_copy(
            emb.at[src_v.at[pl.ds(cb, C)]], rs[b], gsem[b]).wait()
        pltpu.make_async_copy(
            emb.at[dst_v.at[pl.ds(cb, C)]], rd[b], gsem[b]).wait()

    def wissue(k, b):
        cb = k * C
        pltpu.make_async_copy(ps[b], out_s.at[pl.ds(base + cb, C)],
                              wsem[b]).start()
        pltpu.make_async_copy(pd[b], out_d.at[pl.ds(base + cb, C)],
                              wsem[b]).start()

    def wwait(k, b):
        cb = k * C
        pltpu.make_async_copy(ps[b], out_s.at[pl.ds(base + cb, C)],
                              wsem[b]).wait()
        pltpu.make_async_copy(pd[b], out_d.at[pl.ds(base + cb, C)],
                              wsem[b]).wait()

    gissue(0, 0)
    gissue(1, 1)
    gwait(0, 0)
    pack_chunk(0)
    wissue(0, 0)

    def pair(p, _):
        k1 = 2 * p - 1
        wwait(k1 - 1, 0)
        gissue(k1 + 1, 0)
        gwait(k1, 1)
        pack_chunk(1)
        wissue(k1, 1)
        k2 = 2 * p
        wwait(k2 - 1, 1)
        gissue(k2 + 1, 1)
        gwait(k2, 0)
        pack_chunk(0)
        wissue(k2, 0)
        return 0

    lax.fori_loop(1, n_chunks // 2, pair, 0)
    gwait(n_chunks - 1, 1)
    pack_chunk(1)
    wissue(n_chunks - 1, 1)
    wwait(n_chunks - 2, 0)
    wwait(n_chunks - 1, 1)


_ef_kernel = pl.kernel(
    _ef_body,
    out_type=(
        jax.ShapeDtypeStruct((EP, D_OUT // 2), jnp.int32),
        jax.ShapeDtypeStruct((EP, D_OUT // 2), jnp.int32),
    ),
    mesh=_MESH,
    scratch_types=[
        pltpu.VMEM((_E_PER_W,), jnp.int32),
        pltpu.VMEM((_E_PER_W,), jnp.int32),
        pltpu.VMEM((80, D_OUT), jnp.int32),
        pltpu.VMEM((80, D_OUT), jnp.int32),
        pltpu.VMEM((80, D_OUT), jnp.int32),
        pltpu.VMEM((80, D_OUT), jnp.int32),
        pltpu.VMEM((80, D_OUT // 2), jnp.int32),
        pltpu.VMEM((80, D_OUT // 2), jnp.int32),
        pltpu.VMEM((80, D_OUT // 2), jnp.int32),
        pltpu.VMEM((80, D_OUT // 2), jnp.int32),
        pltpu.SemaphoreType.DMA,
        pltpu.SemaphoreType.DMA,
        pltpu.SemaphoreType.DMA,
        pltpu.SemaphoreType.DMA,
    ],
    name="ef_gather",
)


_BN = 1000
_NB = N // _BN


def _dinv_body(deg_ref, raw_ref, dinv_ref, g_ref):
    deg = deg_ref[0, :, 0:1] + deg_ref[1, :, 0:1] + 1.0
    dinv = jnp.where(deg > 0, lax.rsqrt(deg), 0.0)
    dinv_ref[...] = dinv
    g_ref[...] = dinv * raw_ref[...]


def _dinv_kernel(deg, raw):
    return pl.pallas_call(
        _dinv_body,
        grid=(_NB, NC),
        in_specs=[
            pl.BlockSpec((2, _BN, HC), lambda i, c: (0, i, 0)),
            pl.BlockSpec((_BN, D // 2), lambda i, c: (c * _NB + i, 0)),
        ],
        out_specs=(
            pl.BlockSpec((_BN, 1), lambda i, c: (i, 0)),
            pl.BlockSpec((_BN, D // 2), lambda i, c: (c * _NB + i, 0)),
        ),
        out_shape=(
            jax.ShapeDtypeStruct((N, 1), jnp.float32),
            jax.ShapeDtypeStruct((2 * N, D // 2), jnp.float32),
        ),
        name="dinv_scale",
    )(deg, raw)


def _mm1_body(x_ref, w_ref, out_ref):
    out_ref[...] = jnp.dot(x_ref[...], w_ref[...],
                           preferred_element_type=jnp.float32)


def _mm1(x, W1):
    return pl.pallas_call(
        _mm1_body,
        grid=(_NB, NC),
        in_specs=[
            pl.BlockSpec((_BN, D), lambda i, c: (i, 0)),
            pl.BlockSpec((D, D // 2), lambda i, c: (0, c)),
        ],
        out_specs=pl.BlockSpec((_BN, D // 2), lambda i, c: (c * _NB + i, 0)),
        out_shape=jax.ShapeDtypeStruct((2 * N, D // 2), jnp.float32),
        name="mm1",
    )(x, W1)


def _mm_next_body(hw, sA_ref, sB_ref, gA_ref, gB_ref, d_ref, b_ref, w_ref,
                  out_ref):
    d = d_ref[...]
    b = b_ref[...]
    hA = jnp.maximum(d * (sA_ref[0] + gA_ref[...]) + b[:, :hw], 0.0)
    hB = jnp.maximum(d * (sB_ref[0] + gB_ref[...]) + b[:, hw:], 0.0)
    out_ref[...] = d * (
        jnp.dot(hA, w_ref[:hw, :], preferred_element_type=jnp.float32)
        + jnp.dot(hB, w_ref[hw:, :], preferred_element_type=jnp.float32)
    )


def _mm_next(s, g, dinv, b, W, h_in, h_out, split_out=True):
    hw = h_in // 2
    if split_out:
        ow = h_out // 2
        return pl.pallas_call(
            functools.partial(_mm_next_body, hw),
            grid=(_NB, NC),
            in_specs=[
                pl.BlockSpec((1, _BN, hw), lambda i, c: (0, i, 0)),
                pl.BlockSpec((1, _BN, hw), lambda i, c: (1, i, 0)),
                pl.BlockSpec((_BN, hw), lambda i, c: (i, 0)),
                pl.BlockSpec((_BN, hw), lambda i, c: (_NB + i, 0)),
                pl.BlockSpec((_BN, 1), lambda i, c: (i, 0)),
                pl.BlockSpec((1, h_in), lambda i, c: (0, 0)),
                pl.BlockSpec((h_in, ow), lambda i, c: (0, c)),
            ],
            out_specs=pl.BlockSpec((_BN, ow), lambda i, c: (c * _NB + i, 0)),
            out_shape=jax.ShapeDtypeStruct((2 * N, ow), jnp.float32),
            name="mm_next",
        )(s, s, g, g, dinv, b, W)
    return pl.pallas_call(
        functools.partial(_mm_next_body, hw),
        grid=(_NB,),
        in_specs=[
            pl.BlockSpec((1, _BN, hw), lambda i: (0, i, 0)),
            pl.BlockSpec((1, _BN, hw), lambda i: (1, i, 0)),
            pl.BlockSpec((_BN, hw), lambda i: (i, 0)),
            pl.BlockSpec((_BN, hw), lambda i: (_NB + i, 0)),
            pl.BlockSpec((_BN, 1), lambda i: (i, 0)),
            pl.BlockSpec((1, h_in), lambda i: (0, 0)),
            pl.BlockSpec((h_in, h_out), lambda i: (0, 0)),
        ],
        out_specs=pl.BlockSpec((_BN, h_out), lambda i: (i, 0)),
        out_shape=jax.ShapeDtypeStruct((N, h_out), jnp.float32),
        name="mm_next_full",
    )(s, s, g, g, dinv, b, W)


def _emb_body(s0_ref, s1_ref, g_ref, d_ref, b_ref, xout_ref, emb_ref):
    xo = d_ref[...] * (s0_ref[0] + s1_ref[0] + g_ref[...]) + b_ref[...]
    xout_ref[...] = xo
    emb_ref[...] = jnp.maximum(xo, 0.0)


def _emb_kernel(s3, g3, dinv, b3):
    return pl.pallas_call(
        _emb_body,
        grid=(_NB,),
        in_specs=[
            pl.BlockSpec((1, _BN, D_OUT), lambda i: (0, i, 0)),
            pl.BlockSpec((1, _BN, D_OUT), lambda i: (1, i, 0)),
            pl.BlockSpec((_BN, D_OUT), lambda i: (i, 0)),
            pl.BlockSpec((_BN, 1), lambda i: (i, 0)),
            pl.BlockSpec((1, D_OUT), lambda i: (0, 0)),
        ],
        out_specs=(
            pl.BlockSpec((_BN, D_OUT), lambda i: (i, 0)),
            pl.BlockSpec((_BN, D_OUT), lambda i: (i, 0)),
        ),
        out_shape=(
            jax.ShapeDtypeStruct((N, D_OUT), jnp.float32),
            jax.ShapeDtypeStruct((N, D_OUT), jnp.float32),
        ),
        name="emb",
    )(s3, s3, g3, dinv, b3)


_BE = 2560
_EB = -(-E // _BE)


def _mlp_body(efs_ref, efd_ref, w_ref, m1ae_ref, m1ao_ref, m1r_ref,
              m1be_ref, m1bo_ref, mb1_ref, m2_ref, mb2_ref, m3_ref, mb3_ref,
              out_ref):
    bf = jnp.bfloat16
    himask = jnp.int32(-65536)

    def unpack(ref):
        x = ref[...]
        lo = lax.bitcast_convert_type(lax.shift_left(x, 16), jnp.float32)
        hi = lax.bitcast_convert_type(x & himask, jnp.float32)
        return lo.astype(bf), hi.astype(bf)

    s_lo, s_hi = unpack(efs_ref)
    d_lo, d_hi = unpack(efd_ref)
    m = (
        jnp.dot(s_lo, m1ae_ref[...], preferred_element_type=jnp.float32)
        + jnp.dot(s_hi, m1ao_ref[...], preferred_element_type=jnp.float32)
        + jnp.dot(d_lo, m1be_ref[...], preferred_element_type=jnp.float32)
        + jnp.dot(d_hi, m1bo_ref[...], preferred_element_type=jnp.float32)
        + w_ref[...] * m1r_ref[...]
        + mb1_ref[...]
    )
    m = jnp.maximum(m, 0.0)
    m = jnp.dot(m.astype(bf), m2_ref[...],
                preferred_element_type=jnp.float32) + mb2_ref[...]
    m = jnp.maximum(m, 0.0)
    r3 = (
        jnp.dot(m.astype(bf), m3_ref[...],
                preferred_element_type=jnp.float32) + mb3_ref[...]
    )
    out_ref[...] = r3


def _mlp_kernel(efs, efd, ew, M1ae, M1ao, M1r, M1be, M1bo, mb1, M2, mb2,
                M3, mb3):
    def full(shape):
        return pl.BlockSpec(shape, lambda i: tuple(0 for _ in shape))

    out = pl.pallas_call(
        _mlp_body,
        grid=(_EB,),
        in_specs=[
            pl.BlockSpec((_BE, D_OUT // 2), lambda i: (i, 0)),
            pl.BlockSpec((_BE, D_OUT // 2), lambda i: (i, 0)),
            pl.BlockSpec((_BE, 1), lambda i: (i, 0)),
            full((D_OUT // 2, MLP_H)),
            full((D_OUT // 2, MLP_H)),
            full((1, MLP_H)),
            full((D_OUT // 2, MLP_H)),
            full((D_OUT // 2, MLP_H)),
            full((1, MLP_H)),
            full((MLP_H, MLP_H)),
            full((1, MLP_H)),
            full((MLP_H, 2)),
            full((1, 2)),
        ],
        out_specs=pl.BlockSpec((_BE, 2), lambda i: (i, 0)),
        out_shape=jax.ShapeDtypeStruct((E, 2), jnp.float32),
        name="edge_mlp",
    )(efs, efd, ew, M1ae, M1ao, M1r, M1be, M1bo, mb1, M2, mb2, M3, mb3)
    return out


def kernel(x, edge_index, edge_weight, W1, b1, W2, b2, W3, b3,
           M1, mb1, M2, mb2, M3, mb3):
    npad = EP - E
    pad_idx = (jnp.arange(npad, dtype=jnp.int32) * 97) % N
    src = jnp.concatenate([edge_index[0], pad_idx])
    dst = jnp.concatenate([edge_index[1], pad_idx])
    w = jnp.concatenate([edge_weight[:, 0], jnp.zeros((npad,), jnp.float32)])

    deg = _edge_pass_deg(w, src, dst)
    g1raw = _mm1(x, W1)
    dinv, g1 = _dinv_kernel(deg, g1raw)
    s1 = _edge_pass_feat(g1, w, src, dst)
    g2 = _mm_next(s1, g1, dinv, b1.reshape(1, -1), W2, D, D)
    s2 = _edge_pass_feat(g2, w, src, dst)
    g3 = _mm_next(s2, g2, dinv, b2.reshape(1, -1), W3, D, D_OUT,
                  split_out=False)
    s3 = _edge_pass_edge(g3, w, src, dst)
    x_out, emb = _emb_kernel(s3, g3, dinv, b3.reshape(1, -1))

    emb_i = lax.bitcast_convert_type(emb, jnp.int32)
    efs, efd = _ef_kernel(emb_i, src, dst)
    bf = jnp.bfloat16
    perm = []
    for t in range(D_OUT // 32):
        for j in range(16):
            perm.extend([32 * t + j, 32 * t + 16 + j])
    perm = jnp.array(perm, dtype=jnp.int32)
    M1a_p = M1[:D_OUT][perm]
    M1b_p = M1[D_OUT + 1:][perm]
    edge_out = _mlp_kernel(
        efs, efd, edge_weight,
        M1a_p[0::2].astype(bf), M1a_p[1::2].astype(bf), M1[D_OUT:D_OUT + 1],
        M1b_p[0::2].astype(bf), M1b_p[1::2].astype(bf),
        mb1.reshape(1, -1), M2.astype(bf), mb2.reshape(1, -1),
        M3.astype(bf), mb3.reshape(1, -1),
    )
    return (x_out, edge_out)
